# baseline (speedup 1.0000x reference)
import functools

import jax
import jax.numpy as jnp
from jax import lax
from jax.experimental import pallas as pl
from jax.experimental.pallas import tpu as pltpu
from jax.experimental.pallas import tpu_sc as plsc

_K = 256


def _gate_body(x_ref, gw_ref, gb_ref, lg_ref, m_ref, iz_ref, m_scr, z_scr):
    s = pl.program_id(1)
    ns = pl.num_programs(1)
    xb = x_ref[0]
    gw = gw_ref[...]
    lg = lax.dot_general(gw, xb, (((1,), (1,)), ((), ())),
                         preferred_element_type=jnp.float32)
    lg = lg + gb_ref[:, :1]
    lg_ref[:, 0, 0, :] = lg
    bm = jnp.max(lg, axis=1, keepdims=True)
    bz = jnp.sum(jnp.exp(lg - bm), axis=1, keepdims=True)

    @pl.when(s == 0)
    def _init():
        m_scr[...] = jnp.broadcast_to(bm, m_scr.shape)
        z_scr[...] = jnp.broadcast_to(bz, z_scr.shape)

    @pl.when(s != 0)
    def _acc():
        m_old = m_scr[:, :1]
        z_old = z_scr[:, :1]
        m_new = jnp.maximum(m_old, bm)
        z_new = z_old * jnp.exp(m_old - m_new) + bz * jnp.exp(bm - m_new)
        m_scr[...] = jnp.broadcast_to(m_new, m_scr.shape)
        z_scr[...] = jnp.broadcast_to(z_new, z_scr.shape)

    @pl.when(s == ns - 1)
    def _fin():
        m_ref[:, 0, 0, :] = m_scr[...]
        iz_ref[:, 0, 0, :] = 1.0 / z_scr[...]


def _gate(x, gate_w, gate_b):
    B, S, D = x.shape
    E = gate_w.shape[0]
    SB = 1024
    grid = (B, S // SB)
    gb = jnp.broadcast_to(gate_b[:, None], (E, 16))
    return pl.pallas_call(
        _gate_body,
        grid=grid,
        in_specs=[
            pl.BlockSpec((1, SB, D), lambda b, s: (b, s, 0)),
            pl.BlockSpec((E, D), lambda b, s: (0, 0)),
            pl.BlockSpec((E, 16), lambda b, s: (0, 0)),
        ],
        out_specs=[
            pl.BlockSpec((E, 1, 1, SB), lambda b, s: (0, b, 0, s)),
            pl.BlockSpec((E, 1, 1, 16), lambda b, s: (0, b, 0, 0)),
            pl.BlockSpec((E, 1, 1, 16), lambda b, s: (0, b, 0, 0)),
        ],
        out_shape=[
            jax.ShapeDtypeStruct((E, B, 1, S), jnp.float32),
            jax.ShapeDtypeStruct((E, B, 1, 16), jnp.float32),
            jax.ShapeDtypeStruct((E, B, 1, 16), jnp.float32),
        ],
        scratch_shapes=[
            pltpu.VMEM((E, 16), jnp.float32),
            pltpu.VMEM((E, 16), jnp.float32),
        ],
        compiler_params=pltpu.CompilerParams(
            dimension_semantics=("arbitrary", "arbitrary"),
        ),
    )(x, gate_w, gb)


def _thresh_body(lg_ref, t_ref, bud_ref):
    int_min = lax.shift_left(jnp.int32(1), 31)
    bits = lax.bitcast_convert_type(lg_ref[...], jnp.int32)
    skey = bits ^ ((bits >> 31) & jnp.int32(0x7FFFFFFF))

    def step(i, u):
        bit = lax.shift_left(jnp.int32(1), 31 - i)
        ut = u | bit
        ts = ut ^ int_min
        cnt = jnp.sum((skey >= ts).astype(jnp.int32), axis=1, keepdims=True)
        return jnp.where(cnt >= _K, ut, u)

    u0 = jnp.zeros((lg_ref.shape[0], 1), jnp.int32)
    u = lax.fori_loop(0, 32, step, u0)
    ts = u ^ int_min
    g = jnp.sum((skey > ts).astype(jnp.int32), axis=1, keepdims=True)
    bits_t = jnp.where(u < 0, u ^ int_min, ~u)
    t_f = lax.bitcast_convert_type(bits_t, jnp.float32)
    t_ref[...] = jnp.broadcast_to(t_f, t_ref.shape)
    bud_ref[...] = jnp.broadcast_to(_K - g, bud_ref.shape)


def _threshold(lg_r):
    R, S = lg_r.shape
    RB = 8
    grid = (R // RB,)
    return pl.pallas_call(
        _thresh_body,
        grid=grid,
        in_specs=[pl.BlockSpec((RB, S), lambda i: (i, 0))],
        out_specs=[
            pl.BlockSpec((RB, 16), lambda i: (i, 0)),
            pl.BlockSpec((RB, 16), lambda i: (i, 0)),
        ],
        out_shape=[
            jax.ShapeDtypeStruct((R, 16), jnp.float32),
            jax.ShapeDtypeStruct((R, 16), jnp.int32),
        ],
        compiler_params=pltpu.CompilerParams(
            dimension_semantics=("arbitrary",),
        ),
    )(lg_r)


_NW = 32
_RPW = 256 // _NW


def _compact_body(lg, ts, bs, ms, zs, sel_s, sel_p,
                  row_v, t_v, b_v, m_v, z_v, os_v, ol_v, tie_v):
    cid = lax.axis_index("c")
    sid = lax.axis_index("s")
    wid = sid * 2 + cid

    for rr in range(_RPW):
        r = wid * _RPW + rr
        pltpu.sync_copy(lg.at[r], row_v)
        pltpu.sync_copy(ts.at[r], t_v)
        pltpu.sync_copy(bs.at[r], b_v)
        pltpu.sync_copy(ms.at[r], m_v)
        pltpu.sync_copy(zs.at[r], z_v)
        vt = t_v[...]
        vb = b_v[...]

        lanes = lax.iota(jnp.int32, 16)
        one = jnp.ones((16,), jnp.int32)
        zero = jnp.zeros((16,), jnp.int32)

        def step(c, carry):
            off_s, off_t, ii = carry
            v = row_v[pl.ds(c * 16, 16)]
            gt = v > vt
            eq = v == vt
            cs_g = plsc.cumsum(jnp.where(gt, one, zero))
            cs_e = plsc.cumsum(jnp.where(eq, one, zero))
            pos_g = off_s + cs_g - one
            pos_e = off_t + cs_e - one
            plsc.store_scatter(os_v, [pos_g], ii, mask=gt)
            plsc.store_scatter(ol_v, [pos_g], v, mask=gt)
            plsc.store_scatter(tie_v, [pos_e], ii, mask=eq)
            n_g = plsc.all_reduce_population_count(gt)
            n_e = plsc.all_reduce_population_count(eq)
            return off_s + n_g, off_t + n_e, ii + 16 * one

        off_s, _, _ = lax.fori_loop(
            0, 512, step, (zero, zero, lanes))

        nti = (jnp.max(vb) + 15) // 16

        def tstep(t, off):
            tv = tie_v[pl.ds(t * 16, 16)]
            tbase = jnp.full((16,), t * 16, jnp.int32)
            mk = (lanes + tbase) < vb
            cs = plsc.cumsum(jnp.where(mk, one, zero))
            pos = off + cs - one
            plsc.store_scatter(os_v, [pos], tv, mask=mk)
            plsc.store_scatter(ol_v, [pos], vt, mask=mk)
            return off + plsc.all_reduce_population_count(mk)

        lax.fori_loop(0, nti, tstep, off_s)

        vm = m_v[...]
        vz = z_v[...]
        for j in range(_K // 16):
            lv = ol_v[pl.ds(j * 16, 16)]
            ol_v[pl.ds(j * 16, 16)] = jnp.exp(lv - vm) * vz

        pltpu.sync_copy(os_v.at[pl.ds(0, _K)], sel_s.at[r])
        pltpu.sync_copy(ol_v.at[pl.ds(0, _K)], sel_p.at[r])


def _compact(lg_r, t_splat, bud_splat, m_splat, iz_splat):
    R, S = lg_r.shape
    mesh = plsc.VectorSubcoreMesh(core_axis_name="c", subcore_axis_name="s",
                                  num_cores=2, num_subcores=16)
    fn = pl.kernel(
        _compact_body,
        out_type=[
            jax.ShapeDtypeStruct((R, _K), jnp.int32),
            jax.ShapeDtypeStruct((R, _K), jnp.float32),
        ],
        mesh=mesh,
        scratch_types=[
            pltpu.VMEM((S,), jnp.float32),
            pltpu.VMEM((16,), jnp.float32),
            pltpu.VMEM((16,), jnp.int32),
            pltpu.VMEM((16,), jnp.float32),
            pltpu.VMEM((16,), jnp.float32),
            pltpu.VMEM((_K + 16,), jnp.int32),
            pltpu.VMEM((_K + 16,), jnp.float32),
            pltpu.VMEM((S + 16,), jnp.int32),
        ],
        compiler_params=pltpu.CompilerParams(needs_layout_passes=False),
    )
    return fn(lg_r, t_splat, bud_splat, m_splat, iz_splat)


def _mlp_body(inp_ref, w1_ref, b1_ref, w2_ref, b2_ref, vals_ref, out_ref):
    h = pl.program_id(1)
    nh = pl.num_programs(1)
    a = jnp.dot(inp_ref[0], w1_ref[0], preferred_element_type=jnp.float32)
    a = a + b1_ref[0, 0][None, :]
    g = 0.5 * a * (1.0 + jax.lax.erf(a * 0.7071067811865476))
    part = jnp.dot(g, w2_ref[0], preferred_element_type=jnp.float32)

    @pl.when(h == 0)
    def _init():
        out_ref[0] = part

    @pl.when(h != 0)
    def _acc():
        out_ref[0] += part

    @pl.when(h == nh - 1)
    def _fin():
        out_ref[0] = (out_ref[0] + b2_ref[0, 0][None, :]) * vals_ref[0, 0][:, None]


def _mlp(inp, w1a, b1, w2a, b2, vals):
    E, BK, D = inp.shape
    H = w1a.shape[2]
    O = w2a.shape[2]
    HB = 512
    grid = (E, H // HB)
    return pl.pallas_call(
        _mlp_body,
        grid=grid,
        in_specs=[
            pl.BlockSpec((1, BK, D), lambda e, h: (e, 0, 0)),
            pl.BlockSpec((1, D, HB), lambda e, h: (e, 0, h)),
            pl.BlockSpec((1, 1, HB), lambda e, h: (e, 0, h)),
            pl.BlockSpec((1, HB, O), lambda e, h: (e, h, 0)),
            pl.BlockSpec((1, 1, O), lambda e, h: (e, 0, 0)),
            pl.BlockSpec((1, 1, BK), lambda e, h: (e, 0, 0)),
        ],
        out_specs=pl.BlockSpec((1, BK, O), lambda e, h: (e, 0, 0)),
        out_shape=jax.ShapeDtypeStruct((E, BK, O), jnp.float32),
        compiler_params=pltpu.CompilerParams(
            dimension_semantics=("parallel", "arbitrary"),
        ),
    )(inp, w1a, b1, w2a, b2, vals)


def kernel(x, gate_w, gate_b, weight1, weight2):
    B, S, D = x.shape
    E = weight1.shape[0]
    k = _K

    lg, m_s, iz_s = _gate(x, gate_w, gate_b)
    lg_r = lg.reshape(E * B, S)
    t_s, bud_s = _threshold(lg_r)
    sel_s, sel_p = _compact(lg_r, t_s, bud_s,
                            m_s.reshape(E * B, 16), iz_s.reshape(E * B, 16))

    b_of_r = (jnp.arange(E * B, dtype=jnp.int32) % B)[:, None]
    flat_idx = (sel_s + b_of_r * S).reshape(-1)
    inp = x.reshape(B * S, D)[flat_idx].reshape(E, B * k, D)
    valsE = sel_p.reshape(E, 1, B * k)

    w1a = weight1[:, :D, :]
    b1 = weight1[:, D:, :]
    w2a = weight2[:, :-1, :]
    b2 = weight2[:, -1:, :]

    out = _mlp(inp, w1a, b1, w2a, b2, valsE)

    O = out.shape[-1]
    out_b = out.reshape(E, B, k, O).transpose(1, 0, 2, 3).reshape(B, E * k, O)
    scatter_idx = sel_s.reshape(E, B, k).transpose(1, 0, 2).reshape(B, E * k)
    outputs = jnp.zeros((B, S, O), x.dtype).at[
        jnp.arange(B)[:, None], scatter_idx
    ].add(out_b)
    return outputs


# speedup vs baseline: 1.0727x; 1.0727x over previous
import functools

import jax
import jax.numpy as jnp
from jax import lax
from jax.experimental import pallas as pl
from jax.experimental.pallas import tpu as pltpu
from jax.experimental.pallas import tpu_sc as plsc

_K = 256


def _gate_body(x_ref, gw_ref, gb_ref, lg_ref, m_ref, iz_ref, m_scr, z_scr):
    s = pl.program_id(1)
    ns = pl.num_programs(1)
    xb = x_ref[0]
    gw = gw_ref[...]
    lg = lax.dot_general(gw, xb, (((1,), (1,)), ((), ())),
                         preferred_element_type=jnp.float32)
    lg = lg + gb_ref[:, :1]
    lg_ref[:, 0, 0, :] = lg
    bm = jnp.max(lg, axis=1, keepdims=True)
    bz = jnp.sum(jnp.exp(lg - bm), axis=1, keepdims=True)

    @pl.when(s == 0)
    def _init():
        m_scr[...] = jnp.broadcast_to(bm, m_scr.shape)
        z_scr[...] = jnp.broadcast_to(bz, z_scr.shape)

    @pl.when(s != 0)
    def _acc():
        m_old = m_scr[:, :1]
        z_old = z_scr[:, :1]
        m_new = jnp.maximum(m_old, bm)
        z_new = z_old * jnp.exp(m_old - m_new) + bz * jnp.exp(bm - m_new)
        m_scr[...] = jnp.broadcast_to(m_new, m_scr.shape)
        z_scr[...] = jnp.broadcast_to(z_new, z_scr.shape)

    @pl.when(s == ns - 1)
    def _fin():
        m_ref[:, 0, 0, :] = m_scr[...]
        iz_ref[:, 0, 0, :] = 1.0 / z_scr[...]


def _gate(x, gate_w, gate_b):
    B, S, D = x.shape
    E = gate_w.shape[0]
    SB = 1024
    grid = (B, S // SB)
    gb = jnp.broadcast_to(gate_b[:, None], (E, 16))
    return pl.pallas_call(
        _gate_body,
        grid=grid,
        in_specs=[
            pl.BlockSpec((1, SB, D), lambda b, s: (b, s, 0)),
            pl.BlockSpec((E, D), lambda b, s: (0, 0)),
            pl.BlockSpec((E, 16), lambda b, s: (0, 0)),
        ],
        out_specs=[
            pl.BlockSpec((E, 1, 1, SB), lambda b, s: (0, b, 0, s)),
            pl.BlockSpec((E, 1, 1, 16), lambda b, s: (0, b, 0, 0)),
            pl.BlockSpec((E, 1, 1, 16), lambda b, s: (0, b, 0, 0)),
        ],
        out_shape=[
            jax.ShapeDtypeStruct((E, B, 1, S), jnp.float32),
            jax.ShapeDtypeStruct((E, B, 1, 16), jnp.float32),
            jax.ShapeDtypeStruct((E, B, 1, 16), jnp.float32),
        ],
        scratch_shapes=[
            pltpu.VMEM((E, 16), jnp.float32),
            pltpu.VMEM((E, 16), jnp.float32),
        ],
        compiler_params=pltpu.CompilerParams(
            dimension_semantics=("arbitrary", "arbitrary"),
        ),
    )(x, gate_w, gb)


def _thresh_body(lg_ref, t_ref, bud_ref):
    int_min = lax.shift_left(jnp.int32(1), 31)
    bits = lax.bitcast_convert_type(lg_ref[...], jnp.int32)
    skey = bits ^ ((bits >> 31) & jnp.int32(0x7FFFFFFF))

    def step(i, u):
        bit = lax.shift_left(jnp.int32(1), 31 - i)
        ut = u | bit
        ts = ut ^ int_min
        cnt = jnp.sum((skey >= ts).astype(jnp.int32), axis=1, keepdims=True)
        return jnp.where(cnt >= _K, ut, u)

    u0 = jnp.zeros((lg_ref.shape[0], 1), jnp.int32)
    u = lax.fori_loop(0, 32, step, u0)
    ts = u ^ int_min
    g = jnp.sum((skey > ts).astype(jnp.int32), axis=1, keepdims=True)
    bits_t = jnp.where(u < 0, u ^ int_min, ~u)
    t_f = lax.bitcast_convert_type(bits_t, jnp.float32)
    t_ref[...] = jnp.broadcast_to(t_f, t_ref.shape)
    bud_ref[...] = jnp.broadcast_to(_K - g, bud_ref.shape)


def _threshold(lg_r):
    R, S = lg_r.shape
    RB = 8
    grid = (R // RB,)
    return pl.pallas_call(
        _thresh_body,
        grid=grid,
        in_specs=[pl.BlockSpec((RB, S), lambda i: (i, 0))],
        out_specs=[
            pl.BlockSpec((RB, 16), lambda i: (i, 0)),
            pl.BlockSpec((RB, 16), lambda i: (i, 0)),
        ],
        out_shape=[
            jax.ShapeDtypeStruct((R, 16), jnp.float32),
            jax.ShapeDtypeStruct((R, 16), jnp.int32),
        ],
        compiler_params=pltpu.CompilerParams(
            dimension_semantics=("arbitrary",),
        ),
    )(lg_r)


_NW = 32
_RPW = 256 // _NW


def _compact_body(lg, ts, bs, ms, zs, sel_s, sel_p,
                  row_v, t_v, b_v, m_v, z_v, os_v, ol_v, tie_v):
    cid = lax.axis_index("c")
    sid = lax.axis_index("s")
    wid = sid * 2 + cid

    for rr in range(_RPW):
        r = wid * _RPW + rr
        pltpu.sync_copy(lg.at[r], row_v)
        pltpu.sync_copy(ts.at[r], t_v)
        pltpu.sync_copy(bs.at[r], b_v)
        pltpu.sync_copy(ms.at[r], m_v)
        pltpu.sync_copy(zs.at[r], z_v)
        vt = t_v[...]
        vb = b_v[...]

        lanes = lax.iota(jnp.int32, 16)
        one = jnp.ones((16,), jnp.int32)
        zero = jnp.zeros((16,), jnp.int32)

        def step(c, carry):
            off_s, off_t, ii = carry
            v = row_v[pl.ds(c * 16, 16)]
            gt = v > vt
            eq = v == vt
            cs_g = plsc.cumsum(jnp.where(gt, one, zero))
            cs_e = plsc.cumsum(jnp.where(eq, one, zero))
            pos_g = off_s + cs_g - one
            pos_e = off_t + cs_e - one
            plsc.store_scatter(os_v, [pos_g], ii, mask=gt)
            plsc.store_scatter(ol_v, [pos_g], v, mask=gt)
            plsc.store_scatter(tie_v, [pos_e], ii, mask=eq)
            n_g = plsc.all_reduce_population_count(gt)
            n_e = plsc.all_reduce_population_count(eq)
            return off_s + n_g, off_t + n_e, ii + 16 * one

        off_s, _, _ = lax.fori_loop(
            0, 512, step, (zero, zero, lanes))

        nti = (jnp.max(vb) + 15) // 16

        def tstep(t, off):
            tv = tie_v[pl.ds(t * 16, 16)]
            tbase = jnp.full((16,), t * 16, jnp.int32)
            mk = (lanes + tbase) < vb
            cs = plsc.cumsum(jnp.where(mk, one, zero))
            pos = off + cs - one
            plsc.store_scatter(os_v, [pos], tv, mask=mk)
            plsc.store_scatter(ol_v, [pos], vt, mask=mk)
            return off + plsc.all_reduce_population_count(mk)

        lax.fori_loop(0, nti, tstep, off_s)

        vm = m_v[...]
        vz = z_v[...]
        for j in range(_K // 16):
            lv = ol_v[pl.ds(j * 16, 16)]
            ol_v[pl.ds(j * 16, 16)] = jnp.exp(lv - vm) * vz

        pltpu.sync_copy(os_v.at[pl.ds(0, _K)], sel_s.at[r])
        pltpu.sync_copy(ol_v.at[pl.ds(0, _K)], sel_p.at[r])


def _compact(lg_r, t_splat, bud_splat, m_splat, iz_splat):
    R, S = lg_r.shape
    mesh = plsc.VectorSubcoreMesh(core_axis_name="c", subcore_axis_name="s",
                                  num_cores=2, num_subcores=16)
    fn = pl.kernel(
        _compact_body,
        out_type=[
            jax.ShapeDtypeStruct((R, _K), jnp.int32),
            jax.ShapeDtypeStruct((R, _K), jnp.float32),
        ],
        mesh=mesh,
        scratch_types=[
            pltpu.VMEM((S,), jnp.float32),
            pltpu.VMEM((16,), jnp.float32),
            pltpu.VMEM((16,), jnp.int32),
            pltpu.VMEM((16,), jnp.float32),
            pltpu.VMEM((16,), jnp.float32),
            pltpu.VMEM((_K + 16,), jnp.int32),
            pltpu.VMEM((_K + 16,), jnp.float32),
            pltpu.VMEM((S + 16,), jnp.int32),
        ],
        compiler_params=pltpu.CompilerParams(needs_layout_passes=False),
    )
    return fn(lg_r, t_splat, bud_splat, m_splat, iz_splat)


def _mlp_body(inp_ref, w1_ref, b1_ref, w2_ref, b2_ref, vals_ref, out_ref):
    h = pl.program_id(1)
    nh = pl.num_programs(1)
    a = jnp.dot(inp_ref[0].astype(jnp.bfloat16), w1_ref[0],
                preferred_element_type=jnp.float32)
    a = a + b1_ref[0, 0][None, :]
    g = 0.5 * a * (1.0 + jax.lax.erf(a * 0.7071067811865476))
    part = jnp.dot(g.astype(jnp.bfloat16), w2_ref[0],
                   preferred_element_type=jnp.float32)

    @pl.when(h == 0)
    def _init():
        out_ref[0] = part

    @pl.when(h != 0)
    def _acc():
        out_ref[0] += part

    @pl.when(h == nh - 1)
    def _fin():
        out_ref[0] = (out_ref[0] + b2_ref[0, 0][None, :]) * vals_ref[0, 0][:, None]


def _mlp(inp, w1a, b1, w2a, b2, vals):
    E, BK, D = inp.shape
    H = w1a.shape[2]
    O = w2a.shape[2]
    HB = 512
    grid = (E, H // HB)
    w1a = w1a.astype(jnp.bfloat16)
    w2a = w2a.astype(jnp.bfloat16)
    return pl.pallas_call(
        _mlp_body,
        grid=grid,
        in_specs=[
            pl.BlockSpec((1, BK, D), lambda e, h: (e, 0, 0)),
            pl.BlockSpec((1, D, HB), lambda e, h: (e, 0, h)),
            pl.BlockSpec((1, 1, HB), lambda e, h: (e, 0, h)),
            pl.BlockSpec((1, HB, O), lambda e, h: (e, h, 0)),
            pl.BlockSpec((1, 1, O), lambda e, h: (e, 0, 0)),
            pl.BlockSpec((1, 1, BK), lambda e, h: (e, 0, 0)),
        ],
        out_specs=pl.BlockSpec((1, BK, O), lambda e, h: (e, 0, 0)),
        out_shape=jax.ShapeDtypeStruct((E, BK, O), jnp.float32),
        compiler_params=pltpu.CompilerParams(
            dimension_semantics=("parallel", "arbitrary"),
        ),
    )(inp, w1a, b1, w2a, b2, vals)


def kernel(x, gate_w, gate_b, weight1, weight2):
    B, S, D = x.shape
    E = weight1.shape[0]
    k = _K

    lg, m_s, iz_s = _gate(x, gate_w, gate_b)
    lg_r = lg.reshape(E * B, S)
    t_s, bud_s = _threshold(lg_r)
    sel_s, sel_p = _compact(lg_r, t_s, bud_s,
                            m_s.reshape(E * B, 16), iz_s.reshape(E * B, 16))

    b_of_r = (jnp.arange(E * B, dtype=jnp.int32) % B)[:, None]
    flat_idx = (sel_s + b_of_r * S).reshape(-1)
    inp = x.reshape(B * S, D)[flat_idx].reshape(E, B * k, D)
    valsE = sel_p.reshape(E, 1, B * k)

    w1a = weight1[:, :D, :]
    b1 = weight1[:, D:, :]
    w2a = weight2[:, :-1, :]
    b2 = weight2[:, -1:, :]

    out = _mlp(inp, w1a, b1, w2a, b2, valsE)

    O = out.shape[-1]
    out_b = out.reshape(E, B, k, O).transpose(1, 0, 2, 3).reshape(B, E * k, O)
    scatter_idx = sel_s.reshape(E, B, k).transpose(1, 0, 2).reshape(B, E * k)
    outputs = jnp.zeros((B, S, O), x.dtype).at[
        jnp.arange(B)[:, None], scatter_idx
    ].add(out_b)
    return outputs


# speedup vs baseline: 1.2932x; 1.2057x over previous
import functools

import jax
import jax.numpy as jnp
from jax import lax
from jax.experimental import pallas as pl
from jax.experimental.pallas import tpu as pltpu
from jax.experimental.pallas import tpu_sc as plsc

_K = 256


def _gate_body(x_ref, gw_ref, gb_ref, lg_ref, m_ref, iz_ref, m_scr, z_scr):
    s = pl.program_id(1)
    ns = pl.num_programs(1)
    xb = x_ref[0]
    gw = gw_ref[...]
    lg = lax.dot_general(gw, xb, (((1,), (1,)), ((), ())),
                         preferred_element_type=jnp.float32)
    lg = lg + gb_ref[:, :1]
    lg_ref[:, 0, 0, :] = lg
    bm = jnp.max(lg, axis=1, keepdims=True)
    bz = jnp.sum(jnp.exp(lg - bm), axis=1, keepdims=True)

    @pl.when(s == 0)
    def _init():
        m_scr[...] = jnp.broadcast_to(bm, m_scr.shape)
        z_scr[...] = jnp.broadcast_to(bz, z_scr.shape)

    @pl.when(s != 0)
    def _acc():
        m_old = m_scr[:, :1]
        z_old = z_scr[:, :1]
        m_new = jnp.maximum(m_old, bm)
        z_new = z_old * jnp.exp(m_old - m_new) + bz * jnp.exp(bm - m_new)
        m_scr[...] = jnp.broadcast_to(m_new, m_scr.shape)
        z_scr[...] = jnp.broadcast_to(z_new, z_scr.shape)

    @pl.when(s == ns - 1)
    def _fin():
        m_ref[:, 0, 0, :] = m_scr[...]
        iz_ref[:, 0, 0, :] = 1.0 / z_scr[...]


def _gate(x, gate_w, gate_b):
    B, S, D = x.shape
    E = gate_w.shape[0]
    SB = 1024
    grid = (B, S // SB)
    gb = jnp.broadcast_to(gate_b[:, None], (E, 16))
    return pl.pallas_call(
        _gate_body,
        grid=grid,
        in_specs=[
            pl.BlockSpec((1, SB, D), lambda b, s: (b, s, 0)),
            pl.BlockSpec((E, D), lambda b, s: (0, 0)),
            pl.BlockSpec((E, 16), lambda b, s: (0, 0)),
        ],
        out_specs=[
            pl.BlockSpec((E, 1, 1, SB), lambda b, s: (0, b, 0, s)),
            pl.BlockSpec((E, 1, 1, 16), lambda b, s: (0, b, 0, 0)),
            pl.BlockSpec((E, 1, 1, 16), lambda b, s: (0, b, 0, 0)),
        ],
        out_shape=[
            jax.ShapeDtypeStruct((E, B, 1, S), jnp.float32),
            jax.ShapeDtypeStruct((E, B, 1, 16), jnp.float32),
            jax.ShapeDtypeStruct((E, B, 1, 16), jnp.float32),
        ],
        scratch_shapes=[
            pltpu.VMEM((E, 16), jnp.float32),
            pltpu.VMEM((E, 16), jnp.float32),
        ],
        compiler_params=pltpu.CompilerParams(
            dimension_semantics=("arbitrary", "arbitrary"),
        ),
    )(x, gate_w, gb)


def _thresh_body(lg_ref, t_ref, bud_ref):
    int_min = lax.shift_left(jnp.int32(1), 31)
    bits = lax.bitcast_convert_type(lg_ref[...], jnp.int32)
    skey = bits ^ ((bits >> 31) & jnp.int32(0x7FFFFFFF))

    def step(i, u):
        bit = lax.shift_left(jnp.int32(1), 31 - i)
        ut = u | bit
        ts = ut ^ int_min
        cnt = jnp.sum((skey >= ts).astype(jnp.int32), axis=1, keepdims=True)
        return jnp.where(cnt >= _K, ut, u)

    u0 = jnp.zeros((lg_ref.shape[0], 1), jnp.int32)
    u = lax.fori_loop(0, 32, step, u0)
    ts = u ^ int_min
    g = jnp.sum((skey > ts).astype(jnp.int32), axis=1, keepdims=True)
    bits_t = jnp.where(u < 0, u ^ int_min, ~u)
    t_f = lax.bitcast_convert_type(bits_t, jnp.float32)
    t_ref[...] = jnp.broadcast_to(t_f, t_ref.shape)
    bud_ref[...] = jnp.broadcast_to(_K - g, bud_ref.shape)


def _threshold(lg_r):
    R, S = lg_r.shape
    RB = 8
    grid = (R // RB,)
    return pl.pallas_call(
        _thresh_body,
        grid=grid,
        in_specs=[pl.BlockSpec((RB, S), lambda i: (i, 0))],
        out_specs=[
            pl.BlockSpec((RB, 16), lambda i: (i, 0)),
            pl.BlockSpec((RB, 16), lambda i: (i, 0)),
        ],
        out_shape=[
            jax.ShapeDtypeStruct((R, 16), jnp.float32),
            jax.ShapeDtypeStruct((R, 16), jnp.int32),
        ],
        compiler_params=pltpu.CompilerParams(
            dimension_semantics=("arbitrary",),
        ),
    )(lg_r)


_NW = 32
_RPW = 256 // _NW


def _compact_body(lg, ts, bs, ms, zs, x2, sel_s, sel_p, inp,
                  row_v, t_v, b_v, m_v, z_v, os_v, ol_v, tie_v,
                  gi_v, rows_v, sem):
    cid = lax.axis_index("c")
    sid = lax.axis_index("s")
    wid = sid * 2 + cid

    for rr in range(_RPW):
        r = wid * _RPW + rr
        pltpu.sync_copy(lg.at[r], row_v)
        pltpu.sync_copy(ts.at[r], t_v)
        pltpu.sync_copy(bs.at[r], b_v)
        pltpu.sync_copy(ms.at[r], m_v)
        pltpu.sync_copy(zs.at[r], z_v)
        vt = t_v[...]
        vb = b_v[...]

        lanes = lax.iota(jnp.int32, 16)
        one = jnp.ones((16,), jnp.int32)
        zero = jnp.zeros((16,), jnp.int32)

        def step(c, carry):
            off_s, off_t, ii = carry
            v = row_v[pl.ds(c * 16, 16)]
            gt = v > vt
            eq = v == vt
            cs_g = plsc.cumsum(jnp.where(gt, one, zero))
            cs_e = plsc.cumsum(jnp.where(eq, one, zero))
            pos_g = off_s + cs_g - one
            pos_e = off_t + cs_e - one
            plsc.store_scatter(os_v, [pos_g], ii, mask=gt)
            plsc.store_scatter(ol_v, [pos_g], v, mask=gt)
            plsc.store_scatter(tie_v, [pos_e], ii, mask=eq)
            n_g = plsc.all_reduce_population_count(gt)
            n_e = plsc.all_reduce_population_count(eq)
            return off_s + n_g, off_t + n_e, ii + 16 * one

        off_s, _, _ = lax.fori_loop(
            0, 512, step, (zero, zero, lanes))

        nti = (jnp.max(vb) + 15) // 16

        def tstep(t, off):
            tv = tie_v[pl.ds(t * 16, 16)]
            tbase = jnp.full((16,), t * 16, jnp.int32)
            mk = (lanes + tbase) < vb
            cs = plsc.cumsum(jnp.where(mk, one, zero))
            pos = off + cs - one
            plsc.store_scatter(os_v, [pos], tv, mask=mk)
            plsc.store_scatter(ol_v, [pos], vt, mask=mk)
            return off + plsc.all_reduce_population_count(mk)

        lax.fori_loop(0, nti, tstep, off_s)

        vm = m_v[...]
        vz = z_v[...]
        voff = jnp.full((16,), (r % 4) * 8192, jnp.int32)
        for j in range(_K // 16):
            lv = ol_v[pl.ds(j * 16, 16)]
            ol_v[pl.ds(j * 16, 16)] = jnp.exp(lv - vm) * vz
            gi_v[pl.ds(j * 16, 16)] = os_v[pl.ds(j * 16, 16)] + voff

        pltpu.async_copy(x2.at[gi_v], rows_v, sem).wait()
        pltpu.sync_copy(rows_v, inp.at[r])
        pltpu.sync_copy(os_v.at[pl.ds(0, _K)], sel_s.at[r])
        pltpu.sync_copy(ol_v.at[pl.ds(0, _K)], sel_p.at[r])


def _compact(lg_r, t_splat, bud_splat, m_splat, iz_splat, x2):
    R, S = lg_r.shape
    D = x2.shape[1]
    mesh = plsc.VectorSubcoreMesh(core_axis_name="c", subcore_axis_name="s",
                                  num_cores=2, num_subcores=16)
    fn = pl.kernel(
        _compact_body,
        out_type=[
            jax.ShapeDtypeStruct((R, _K), jnp.int32),
            jax.ShapeDtypeStruct((R, _K), jnp.float32),
            jax.ShapeDtypeStruct((R, _K, D), jnp.float32),
        ],
        mesh=mesh,
        scratch_types=[
            pltpu.VMEM((S,), jnp.float32),
            pltpu.VMEM((16,), jnp.float32),
            pltpu.VMEM((16,), jnp.int32),
            pltpu.VMEM((16,), jnp.float32),
            pltpu.VMEM((16,), jnp.float32),
            pltpu.VMEM((_K + 16,), jnp.int32),
            pltpu.VMEM((_K + 16,), jnp.float32),
            pltpu.VMEM((S + 16,), jnp.int32),
            pltpu.VMEM((_K,), jnp.int32),
            pltpu.VMEM((_K, D), jnp.float32),
            pltpu.SemaphoreType.DMA,
        ],
        compiler_params=pltpu.CompilerParams(needs_layout_passes=False),
    )
    return fn(lg_r, t_splat, bud_splat, m_splat, iz_splat, x2)


def _mlp_body(inp_ref, w1_ref, b1_ref, w2_ref, b2_ref, vals_ref, out_ref):
    h = pl.program_id(1)
    nh = pl.num_programs(1)
    a = jnp.dot(inp_ref[0].astype(jnp.bfloat16), w1_ref[0],
                preferred_element_type=jnp.float32)
    a = a + b1_ref[0, 0][None, :]
    g = 0.5 * a * (1.0 + jax.lax.erf(a * 0.7071067811865476))
    part = jnp.dot(g.astype(jnp.bfloat16), w2_ref[0],
                   preferred_element_type=jnp.float32)

    @pl.when(h == 0)
    def _init():
        out_ref[0] = part

    @pl.when(h != 0)
    def _acc():
        out_ref[0] += part

    @pl.when(h == nh - 1)
    def _fin():
        out_ref[0] = (out_ref[0] + b2_ref[0, 0][None, :]) * vals_ref[0, 0][:, None]


def _mlp(inp, w1a, b1, w2a, b2, vals):
    E, BK, D = inp.shape
    H = w1a.shape[2]
    O = w2a.shape[2]
    HB = 512
    grid = (E, H // HB)
    w1a = w1a.astype(jnp.bfloat16)
    w2a = w2a.astype(jnp.bfloat16)
    return pl.pallas_call(
        _mlp_body,
        grid=grid,
        in_specs=[
            pl.BlockSpec((1, BK, D), lambda e, h: (e, 0, 0)),
            pl.BlockSpec((1, D, HB), lambda e, h: (e, 0, h)),
            pl.BlockSpec((1, 1, HB), lambda e, h: (e, 0, h)),
            pl.BlockSpec((1, HB, O), lambda e, h: (e, h, 0)),
            pl.BlockSpec((1, 1, O), lambda e, h: (e, 0, 0)),
            pl.BlockSpec((1, 1, BK), lambda e, h: (e, 0, 0)),
        ],
        out_specs=pl.BlockSpec((1, BK, O), lambda e, h: (e, 0, 0)),
        out_shape=jax.ShapeDtypeStruct((E, BK, O), jnp.float32),
        compiler_params=pltpu.CompilerParams(
            dimension_semantics=("parallel", "arbitrary"),
        ),
    )(inp, w1a, b1, w2a, b2, vals)


def kernel(x, gate_w, gate_b, weight1, weight2):
    B, S, D = x.shape
    E = weight1.shape[0]
    k = _K

    lg, m_s, iz_s = _gate(x, gate_w, gate_b)
    lg_r = lg.reshape(E * B, S)
    t_s, bud_s = _threshold(lg_r)
    sel_s, sel_p, inp = _compact(lg_r, t_s, bud_s,
                                 m_s.reshape(E * B, 16),
                                 iz_s.reshape(E * B, 16),
                                 x.reshape(B * S, D))
    inp = inp.reshape(E, B * k, D)
    valsE = sel_p.reshape(E, 1, B * k)

    w1a = weight1[:, :D, :]
    b1 = weight1[:, D:, :]
    w2a = weight2[:, :-1, :]
    b2 = weight2[:, -1:, :]

    out = _mlp(inp, w1a, b1, w2a, b2, valsE)

    O = out.shape[-1]
    out_b = out.reshape(E, B, k, O).transpose(1, 0, 2, 3).reshape(B, E * k, O)
    scatter_idx = sel_s.reshape(E, B, k).transpose(1, 0, 2).reshape(B, E * k)
    outputs = jnp.zeros((B, S, O), x.dtype).at[
        jnp.arange(B)[:, None], scatter_idx
    ].add(out_b)
    return outputs


# speedup vs baseline: 1.6680x; 1.2898x over previous
import functools

import jax
import jax.numpy as jnp
from jax import lax
from jax.experimental import pallas as pl
from jax.experimental.pallas import tpu as pltpu
from jax.experimental.pallas import tpu_sc as plsc

_K = 256


def _gate_body(x_ref, gw_ref, gb_ref, lg_ref, m_ref, iz_ref, m_scr, z_scr):
    s = pl.program_id(1)
    ns = pl.num_programs(1)
    xb = x_ref[0]
    gw = gw_ref[...]
    lg = lax.dot_general(gw, xb, (((1,), (1,)), ((), ())),
                         preferred_element_type=jnp.float32)
    lg = lg + gb_ref[:, :1]
    lg_ref[:, 0, 0, :] = lg
    bm = jnp.max(lg, axis=1, keepdims=True)
    bz = jnp.sum(jnp.exp(lg - bm), axis=1, keepdims=True)

    @pl.when(s == 0)
    def _init():
        m_scr[...] = jnp.broadcast_to(bm, m_scr.shape)
        z_scr[...] = jnp.broadcast_to(bz, z_scr.shape)

    @pl.when(s != 0)
    def _acc():
        m_old = m_scr[:, :1]
        z_old = z_scr[:, :1]
        m_new = jnp.maximum(m_old, bm)
        z_new = z_old * jnp.exp(m_old - m_new) + bz * jnp.exp(bm - m_new)
        m_scr[...] = jnp.broadcast_to(m_new, m_scr.shape)
        z_scr[...] = jnp.broadcast_to(z_new, z_scr.shape)

    @pl.when(s == ns - 1)
    def _fin():
        m_ref[:, 0, 0, :] = m_scr[...]
        iz_ref[:, 0, 0, :] = 1.0 / z_scr[...]


def _gate(x, gate_w, gate_b):
    B, S, D = x.shape
    E = gate_w.shape[0]
    SB = 1024
    grid = (B, S // SB)
    gb = jnp.broadcast_to(gate_b[:, None], (E, 16))
    return pl.pallas_call(
        _gate_body,
        grid=grid,
        in_specs=[
            pl.BlockSpec((1, SB, D), lambda b, s: (b, s, 0)),
            pl.BlockSpec((E, D), lambda b, s: (0, 0)),
            pl.BlockSpec((E, 16), lambda b, s: (0, 0)),
        ],
        out_specs=[
            pl.BlockSpec((E, 1, 1, SB), lambda b, s: (0, b, 0, s)),
            pl.BlockSpec((E, 1, 1, 16), lambda b, s: (0, b, 0, 0)),
            pl.BlockSpec((E, 1, 1, 16), lambda b, s: (0, b, 0, 0)),
        ],
        out_shape=[
            jax.ShapeDtypeStruct((E, B, 1, S), jnp.float32),
            jax.ShapeDtypeStruct((E, B, 1, 16), jnp.float32),
            jax.ShapeDtypeStruct((E, B, 1, 16), jnp.float32),
        ],
        scratch_shapes=[
            pltpu.VMEM((E, 16), jnp.float32),
            pltpu.VMEM((E, 16), jnp.float32),
        ],
        compiler_params=pltpu.CompilerParams(
            dimension_semantics=("arbitrary", "arbitrary"),
        ),
    )(x, gate_w, gb)


def _thresh_body(lg_ref, t_ref, bud_ref):
    int_min = lax.shift_left(jnp.int32(1), 31)
    bits = lax.bitcast_convert_type(lg_ref[...], jnp.int32)
    skey = bits ^ ((bits >> 31) & jnp.int32(0x7FFFFFFF))

    def step(i, u):
        bit = lax.shift_left(jnp.int32(1), 31 - i)
        ut = u | bit
        ts = ut ^ int_min
        cnt = jnp.sum((skey >= ts).astype(jnp.int32), axis=1, keepdims=True)
        return jnp.where(cnt >= _K, ut, u)

    u0 = jnp.zeros((lg_ref.shape[0], 1), jnp.int32)
    u = lax.fori_loop(0, 32, step, u0)
    ts = u ^ int_min
    g = jnp.sum((skey > ts).astype(jnp.int32), axis=1, keepdims=True)
    bits_t = jnp.where(u < 0, u ^ int_min, ~u)
    t_f = lax.bitcast_convert_type(bits_t, jnp.float32)
    t_ref[...] = jnp.broadcast_to(t_f, t_ref.shape)
    bud_ref[...] = jnp.broadcast_to(_K - g, bud_ref.shape)


def _threshold(lg_r):
    R, S = lg_r.shape
    RB = 8
    grid = (R // RB,)
    return pl.pallas_call(
        _thresh_body,
        grid=grid,
        in_specs=[pl.BlockSpec((RB, S), lambda i: (i, 0))],
        out_specs=[
            pl.BlockSpec((RB, 16), lambda i: (i, 0)),
            pl.BlockSpec((RB, 16), lambda i: (i, 0)),
        ],
        out_shape=[
            jax.ShapeDtypeStruct((R, 16), jnp.float32),
            jax.ShapeDtypeStruct((R, 16), jnp.int32),
        ],
        compiler_params=pltpu.CompilerParams(
            dimension_semantics=("arbitrary",),
        ),
    )(lg_r)


_NW = 32
_RPW = 256 // _NW


def _compact_body(lg, ts, bs, ms, zs, x2, sel_s, sel_p, inp,
                  row_v, t_v, b_v, m_v, z_v, os_v, ol_v, tie_v,
                  gi_v, rows_v, sem):
    cid = lax.axis_index("c")
    sid = lax.axis_index("s")
    wid = sid * 2 + cid

    for rr in range(_RPW):
        r = wid * _RPW + rr
        pltpu.sync_copy(lg.at[r], row_v)
        pltpu.sync_copy(ts.at[r], t_v)
        pltpu.sync_copy(bs.at[r], b_v)
        pltpu.sync_copy(ms.at[r], m_v)
        pltpu.sync_copy(zs.at[r], z_v)
        vt = t_v[...]
        vb = b_v[...]

        lanes = lax.iota(jnp.int32, 16)
        one = jnp.ones((16,), jnp.int32)
        zero = jnp.zeros((16,), jnp.int32)

        def step(c, carry):
            off_s, off_t, ii = carry
            v = row_v[pl.ds(c * 16, 16)]
            gt = v > vt
            eq = v == vt
            cs_g = plsc.cumsum(jnp.where(gt, one, zero))
            cs_e = plsc.cumsum(jnp.where(eq, one, zero))
            pos_g = off_s + cs_g - one
            pos_e = off_t + cs_e - one
            plsc.store_scatter(os_v, [pos_g], ii, mask=gt)
            plsc.store_scatter(ol_v, [pos_g], v, mask=gt)
            plsc.store_scatter(tie_v, [pos_e], ii, mask=eq)
            n_g = plsc.all_reduce_population_count(gt)
            n_e = plsc.all_reduce_population_count(eq)
            return off_s + n_g, off_t + n_e, ii + 16 * one

        off_s, _, _ = lax.fori_loop(
            0, 512, step, (zero, zero, lanes))

        nti = (jnp.max(vb) + 15) // 16

        def tstep(t, off):
            tv = tie_v[pl.ds(t * 16, 16)]
            tbase = jnp.full((16,), t * 16, jnp.int32)
            mk = (lanes + tbase) < vb
            cs = plsc.cumsum(jnp.where(mk, one, zero))
            pos = off + cs - one
            plsc.store_scatter(os_v, [pos], tv, mask=mk)
            plsc.store_scatter(ol_v, [pos], vt, mask=mk)
            return off + plsc.all_reduce_population_count(mk)

        lax.fori_loop(0, nti, tstep, off_s)

        vm = m_v[...]
        vz = z_v[...]
        voff = jnp.full((16,), (r % 4) * 8192, jnp.int32)
        for j in range(_K // 16):
            lv = ol_v[pl.ds(j * 16, 16)]
            ol_v[pl.ds(j * 16, 16)] = jnp.exp(lv - vm) * vz
            gi_v[pl.ds(j * 16, 16)] = os_v[pl.ds(j * 16, 16)] + voff

        pltpu.async_copy(x2.at[gi_v], rows_v, sem).wait()
        pltpu.sync_copy(rows_v, inp.at[r])
        pltpu.sync_copy(os_v.at[pl.ds(0, _K)], sel_s.at[r])
        pltpu.sync_copy(ol_v.at[pl.ds(0, _K)], sel_p.at[r])


def _compact(lg_r, t_splat, bud_splat, m_splat, iz_splat, x2):
    R, S = lg_r.shape
    D = x2.shape[1]
    mesh = plsc.VectorSubcoreMesh(core_axis_name="c", subcore_axis_name="s",
                                  num_cores=2, num_subcores=16)
    fn = pl.kernel(
        _compact_body,
        out_type=[
            jax.ShapeDtypeStruct((R, _K), jnp.int32),
            jax.ShapeDtypeStruct((R, _K), jnp.float32),
            jax.ShapeDtypeStruct((R, _K, D), jnp.float32),
        ],
        mesh=mesh,
        scratch_types=[
            pltpu.VMEM((S,), jnp.float32),
            pltpu.VMEM((16,), jnp.float32),
            pltpu.VMEM((16,), jnp.int32),
            pltpu.VMEM((16,), jnp.float32),
            pltpu.VMEM((16,), jnp.float32),
            pltpu.VMEM((_K + 16,), jnp.int32),
            pltpu.VMEM((_K + 16,), jnp.float32),
            pltpu.VMEM((S + 16,), jnp.int32),
            pltpu.VMEM((_K,), jnp.int32),
            pltpu.VMEM((_K, D), jnp.float32),
            pltpu.SemaphoreType.DMA,
        ],
        compiler_params=pltpu.CompilerParams(needs_layout_passes=False),
    )
    return fn(lg_r, t_splat, bud_splat, m_splat, iz_splat, x2)


def _scatter_body(rows, sel, zeros, out, ch_v, si_v, sh_v):
    cid = lax.axis_index("c")
    sid = lax.axis_index("s")

    for bb in range(2):
        batch = cid * 2 + bb
        pltpu.sync_copy(zeros.at[pl.ds(sid * 512, 512)],
                        sh_v.at[pl.ds(sid * 512, 512)])
        plsc.subcore_barrier()
        for i in range(4):
            e = sid + 16 * i
            r = e * 4 + batch
            pltpu.sync_copy(rows.at[r], ch_v)
            pltpu.sync_copy(sel.at[r], si_v)
            pltpu.sync_copy(ch_v, sh_v.at[si_v], add=True)
        plsc.subcore_barrier()
        pltpu.sync_copy(sh_v.at[pl.ds(sid * 512, 512)],
                        out.at[batch, pl.ds(sid * 512, 512)])
        plsc.subcore_barrier()


def _scatter(rows, sel, B, S):
    R, K, O = rows.shape
    mesh = plsc.VectorSubcoreMesh(core_axis_name="c", subcore_axis_name="s",
                                  num_cores=2, num_subcores=16)
    fn = pl.kernel(
        _scatter_body,
        out_type=jax.ShapeDtypeStruct((B, S, O), jnp.float32),
        mesh=mesh,
        scratch_types=[
            pltpu.VMEM((K, O), jnp.float32),
            pltpu.VMEM((K,), jnp.int32),
            pltpu.VMEM_SHARED((S, O), jnp.float32),
        ],
        compiler_params=pltpu.CompilerParams(needs_layout_passes=False),
    )
    return fn(rows, sel, jnp.zeros((S, O), jnp.float32))


def _mlp_body(inp_ref, w1_ref, b1_ref, w2_ref, b2_ref, vals_ref, out_ref):
    h = pl.program_id(1)
    nh = pl.num_programs(1)
    a = jnp.dot(inp_ref[0].astype(jnp.bfloat16), w1_ref[0],
                preferred_element_type=jnp.float32)
    a = a + b1_ref[0, 0][None, :]
    g = 0.5 * a * (1.0 + jax.lax.erf(a * 0.7071067811865476))
    part = jnp.dot(g.astype(jnp.bfloat16), w2_ref[0],
                   preferred_element_type=jnp.float32)

    @pl.when(h == 0)
    def _init():
        out_ref[0] = part

    @pl.when(h != 0)
    def _acc():
        out_ref[0] += part

    @pl.when(h == nh - 1)
    def _fin():
        out_ref[0] = (out_ref[0] + b2_ref[0, 0][None, :]) * vals_ref[0, 0][:, None]


def _mlp(inp, w1a, b1, w2a, b2, vals):
    E, BK, D = inp.shape
    H = w1a.shape[2]
    O = w2a.shape[2]
    HB = 512
    grid = (E, H // HB)
    w1a = w1a.astype(jnp.bfloat16)
    w2a = w2a.astype(jnp.bfloat16)
    return pl.pallas_call(
        _mlp_body,
        grid=grid,
        in_specs=[
            pl.BlockSpec((1, BK, D), lambda e, h: (e, 0, 0)),
            pl.BlockSpec((1, D, HB), lambda e, h: (e, 0, h)),
            pl.BlockSpec((1, 1, HB), lambda e, h: (e, 0, h)),
            pl.BlockSpec((1, HB, O), lambda e, h: (e, h, 0)),
            pl.BlockSpec((1, 1, O), lambda e, h: (e, 0, 0)),
            pl.BlockSpec((1, 1, BK), lambda e, h: (e, 0, 0)),
        ],
        out_specs=pl.BlockSpec((1, BK, O), lambda e, h: (e, 0, 0)),
        out_shape=jax.ShapeDtypeStruct((E, BK, O), jnp.float32),
        compiler_params=pltpu.CompilerParams(
            dimension_semantics=("parallel", "arbitrary"),
        ),
    )(inp, w1a, b1, w2a, b2, vals)


def kernel(x, gate_w, gate_b, weight1, weight2):
    B, S, D = x.shape
    E = weight1.shape[0]
    k = _K

    lg, m_s, iz_s = _gate(x, gate_w, gate_b)
    lg_r = lg.reshape(E * B, S)
    t_s, bud_s = _threshold(lg_r)
    sel_s, sel_p, inp = _compact(lg_r, t_s, bud_s,
                                 m_s.reshape(E * B, 16),
                                 iz_s.reshape(E * B, 16),
                                 x.reshape(B * S, D))
    inp = inp.reshape(E, B * k, D)
    valsE = sel_p.reshape(E, 1, B * k)

    w1a = weight1[:, :D, :]
    b1 = weight1[:, D:, :]
    w2a = weight2[:, :-1, :]
    b2 = weight2[:, -1:, :]

    out = _mlp(inp, w1a, b1, w2a, b2, valsE)

    O = out.shape[-1]
    return _scatter(out.reshape(E * B, k, O), sel_s, B, S)


# speedup vs baseline: 1.8250x; 1.0941x over previous
import functools

import jax
import jax.numpy as jnp
from jax import lax
from jax.experimental import pallas as pl
from jax.experimental.pallas import tpu as pltpu
from jax.experimental.pallas import tpu_sc as plsc

_K = 256


def _gate_body(x_ref, gw_ref, gb_ref, lg_ref, m_ref, iz_ref, m_scr, z_scr):
    s = pl.program_id(1)
    ns = pl.num_programs(1)
    xb = x_ref[0]
    gw = gw_ref[...]
    lg = lax.dot_general(gw, xb, (((1,), (1,)), ((), ())),
                         preferred_element_type=jnp.float32)
    lg = lg + gb_ref[:, :1]
    lg_ref[:, 0, 0, :] = lg
    bm = jnp.max(lg, axis=1, keepdims=True)
    bz = jnp.sum(jnp.exp(lg - bm), axis=1, keepdims=True)

    @pl.when(s == 0)
    def _init():
        m_scr[...] = jnp.broadcast_to(bm, m_scr.shape)
        z_scr[...] = jnp.broadcast_to(bz, z_scr.shape)

    @pl.when(s != 0)
    def _acc():
        m_old = m_scr[:, :1]
        z_old = z_scr[:, :1]
        m_new = jnp.maximum(m_old, bm)
        z_new = z_old * jnp.exp(m_old - m_new) + bz * jnp.exp(bm - m_new)
        m_scr[...] = jnp.broadcast_to(m_new, m_scr.shape)
        z_scr[...] = jnp.broadcast_to(z_new, z_scr.shape)

    @pl.when(s == ns - 1)
    def _fin():
        m_ref[:, 0, 0, :] = m_scr[...]
        iz_ref[:, 0, 0, :] = 1.0 / z_scr[...]


def _gate(x, gate_w, gate_b):
    B, S, D = x.shape
    E = gate_w.shape[0]
    SB = 1024
    grid = (B, S // SB)
    gb = jnp.broadcast_to(gate_b[:, None], (E, 16))
    return pl.pallas_call(
        _gate_body,
        grid=grid,
        in_specs=[
            pl.BlockSpec((1, SB, D), lambda b, s: (b, s, 0)),
            pl.BlockSpec((E, D), lambda b, s: (0, 0)),
            pl.BlockSpec((E, 16), lambda b, s: (0, 0)),
        ],
        out_specs=[
            pl.BlockSpec((E, 1, 1, SB), lambda b, s: (0, b, 0, s)),
            pl.BlockSpec((E, 1, 1, 16), lambda b, s: (0, b, 0, 0)),
            pl.BlockSpec((E, 1, 1, 16), lambda b, s: (0, b, 0, 0)),
        ],
        out_shape=[
            jax.ShapeDtypeStruct((E, B, 1, S), jnp.float32),
            jax.ShapeDtypeStruct((E, B, 1, 16), jnp.float32),
            jax.ShapeDtypeStruct((E, B, 1, 16), jnp.float32),
        ],
        scratch_shapes=[
            pltpu.VMEM((E, 16), jnp.float32),
            pltpu.VMEM((E, 16), jnp.float32),
        ],
        compiler_params=pltpu.CompilerParams(
            dimension_semantics=("arbitrary", "arbitrary"),
        ),
    )(x, gate_w, gb)


def _thresh_body(lg_ref, t_ref, bud_ref):
    int_min = lax.shift_left(jnp.int32(1), 31)
    bits = lax.bitcast_convert_type(lg_ref[...], jnp.int32)
    skey = bits ^ ((bits >> 31) & jnp.int32(0x7FFFFFFF))

    def step(i, u):
        bit = lax.shift_left(jnp.int32(1), 31 - i)
        ut = u | bit
        ts = ut ^ int_min
        cnt = jnp.sum((skey >= ts).astype(jnp.int32), axis=1, keepdims=True)
        return jnp.where(cnt >= _K, ut, u)

    u0 = jnp.zeros((lg_ref.shape[0], 1), jnp.int32)
    u = lax.fori_loop(0, 32, step, u0)
    ts = u ^ int_min
    g = jnp.sum((skey > ts).astype(jnp.int32), axis=1, keepdims=True)
    bits_t = jnp.where(u < 0, u ^ int_min, ~u)
    t_f = lax.bitcast_convert_type(bits_t, jnp.float32)
    t_ref[...] = jnp.broadcast_to(t_f, t_ref.shape)
    bud_ref[...] = jnp.broadcast_to(_K - g, bud_ref.shape)


def _threshold(lg_r):
    R, S = lg_r.shape
    RB = 8
    grid = (R // RB,)
    return pl.pallas_call(
        _thresh_body,
        grid=grid,
        in_specs=[pl.BlockSpec((RB, S), lambda i: (i, 0))],
        out_specs=[
            pl.BlockSpec((RB, 16), lambda i: (i, 0)),
            pl.BlockSpec((RB, 16), lambda i: (i, 0)),
        ],
        out_shape=[
            jax.ShapeDtypeStruct((R, 16), jnp.float32),
            jax.ShapeDtypeStruct((R, 16), jnp.int32),
        ],
        compiler_params=pltpu.CompilerParams(
            dimension_semantics=("arbitrary",),
        ),
    )(lg_r)


_NW = 32
_RPW = 256 // _NW


def _compact_body(lg, ts, bs, ms, zs, x2, sel_s, sel_p, inp,
                  row_v, t_v, b_v, m_v, z_v, os_v, ol_v, tie_v,
                  gi_v, rows_v, sem):
    cid = lax.axis_index("c")
    sid = lax.axis_index("s")
    wid = sid * 2 + cid

    for rr in range(_RPW):
        r = wid * _RPW + rr
        pltpu.sync_copy(lg.at[r], row_v)
        pltpu.sync_copy(ts.at[r], t_v)
        pltpu.sync_copy(bs.at[r], b_v)
        pltpu.sync_copy(ms.at[r], m_v)
        pltpu.sync_copy(zs.at[r], z_v)
        vt = t_v[...]
        vb = b_v[...]

        lanes = lax.iota(jnp.int32, 16)
        one = jnp.ones((16,), jnp.int32)
        zero = jnp.zeros((16,), jnp.int32)

        def step(c, carry):
            off_s, off_t, ii = carry
            v = row_v[pl.ds(c * 16, 16)]
            gt = v > vt
            eq = v == vt
            cs_g = plsc.cumsum(jnp.where(gt, one, zero))
            cs_e = plsc.cumsum(jnp.where(eq, one, zero))
            pos_g = off_s + cs_g - one
            pos_e = off_t + cs_e - one
            plsc.store_scatter(os_v, [pos_g], ii, mask=gt)
            plsc.store_scatter(ol_v, [pos_g], v, mask=gt)
            plsc.store_scatter(tie_v, [pos_e], ii, mask=eq)
            n_g = plsc.all_reduce_population_count(gt)
            n_e = plsc.all_reduce_population_count(eq)
            return off_s + n_g, off_t + n_e, ii + 16 * one

        off_s, _, _ = lax.fori_loop(
            0, 512, step, (zero, zero, lanes))

        nti = (jnp.max(vb) + 15) // 16

        def tstep(t, off):
            tv = tie_v[pl.ds(t * 16, 16)]
            tbase = jnp.full((16,), t * 16, jnp.int32)
            mk = (lanes + tbase) < vb
            cs = plsc.cumsum(jnp.where(mk, one, zero))
            pos = off + cs - one
            plsc.store_scatter(os_v, [pos], tv, mask=mk)
            plsc.store_scatter(ol_v, [pos], vt, mask=mk)
            return off + plsc.all_reduce_population_count(mk)

        lax.fori_loop(0, nti, tstep, off_s)

        vm = m_v[...]
        vz = z_v[...]
        voff = jnp.full((16,), (r % 4) * 8192, jnp.int32)
        for j in range(_K // 16):
            lv = ol_v[pl.ds(j * 16, 16)]
            ol_v[pl.ds(j * 16, 16)] = jnp.exp(lv - vm) * vz
            gi_v[pl.ds(j * 16, 16)] = os_v[pl.ds(j * 16, 16)] + voff

        pltpu.async_copy(x2.at[gi_v], rows_v, sem).wait()
        pltpu.sync_copy(rows_v, inp.at[r])
        pltpu.sync_copy(os_v.at[pl.ds(0, _K)], sel_s.at[r])
        pltpu.sync_copy(ol_v.at[pl.ds(0, _K)], sel_p.at[r])


def _compact(lg_r, t_splat, bud_splat, m_splat, iz_splat, x2):
    R, S = lg_r.shape
    D = x2.shape[1]
    mesh = plsc.VectorSubcoreMesh(core_axis_name="c", subcore_axis_name="s",
                                  num_cores=2, num_subcores=16)
    fn = pl.kernel(
        _compact_body,
        out_type=[
            jax.ShapeDtypeStruct((R, _K), jnp.int32),
            jax.ShapeDtypeStruct((R, _K), jnp.float32),
            jax.ShapeDtypeStruct((R, _K, D), jnp.float32),
        ],
        mesh=mesh,
        scratch_types=[
            pltpu.VMEM((S,), jnp.float32),
            pltpu.VMEM((16,), jnp.float32),
            pltpu.VMEM((16,), jnp.int32),
            pltpu.VMEM((16,), jnp.float32),
            pltpu.VMEM((16,), jnp.float32),
            pltpu.VMEM((_K + 16,), jnp.int32),
            pltpu.VMEM((_K + 16,), jnp.float32),
            pltpu.VMEM((S + 16,), jnp.int32),
            pltpu.VMEM((_K,), jnp.int32),
            pltpu.VMEM((_K, D), jnp.float32),
            pltpu.SemaphoreType.DMA,
        ],
        compiler_params=pltpu.CompilerParams(needs_layout_passes=False),
    )
    return fn(lg_r, t_splat, bud_splat, m_splat, iz_splat, x2)


def _scatter_body(rows, sel, zeros, out, ch_v, si_v, sh_v):
    cid = lax.axis_index("c")
    sid = lax.axis_index("s")

    for bb in range(2):
        batch = cid * 2 + bb
        pltpu.sync_copy(zeros.at[pl.ds(sid * 512, 512)],
                        sh_v.at[pl.ds(sid * 512, 512)])
        plsc.subcore_barrier()
        for i in range(4):
            e = sid + 16 * i
            r = e * 4 + batch
            pltpu.sync_copy(rows.at[r], ch_v)
            pltpu.sync_copy(sel.at[r], si_v)
            pltpu.sync_copy(ch_v, sh_v.at[si_v], add=True)
        plsc.subcore_barrier()
        pltpu.sync_copy(sh_v.at[pl.ds(sid * 512, 512)],
                        out.at[batch, pl.ds(sid * 512, 512)])
        plsc.subcore_barrier()


def _scatter(rows, sel, B, S):
    R, K, O = rows.shape
    mesh = plsc.VectorSubcoreMesh(core_axis_name="c", subcore_axis_name="s",
                                  num_cores=2, num_subcores=16)
    fn = pl.kernel(
        _scatter_body,
        out_type=jax.ShapeDtypeStruct((B, S, O), jnp.float32),
        mesh=mesh,
        scratch_types=[
            pltpu.VMEM((K, O), jnp.float32),
            pltpu.VMEM((K,), jnp.int32),
            pltpu.VMEM_SHARED((S, O), jnp.float32),
        ],
        compiler_params=pltpu.CompilerParams(needs_layout_passes=False),
    )
    return fn(rows, sel, jnp.zeros((S, O), jnp.float32))


def _mlp_body(inp_ref, w1_ref, b1_ref, w2_ref, b2_ref, vals_ref, out_ref):
    h = pl.program_id(1)
    nh = pl.num_programs(1)
    a = jnp.dot(inp_ref[0].astype(jnp.bfloat16), w1_ref[0],
                preferred_element_type=jnp.float32)
    a = a + b1_ref[0, 0][None, :]
    g = 0.5 * a * (1.0 + jax.lax.erf(a * 0.7071067811865476))
    part = jnp.dot(g.astype(jnp.bfloat16), w2_ref[0],
                   preferred_element_type=jnp.float32)

    @pl.when(h == 0)
    def _init():
        out_ref[0] = part

    @pl.when(h != 0)
    def _acc():
        out_ref[0] += part

    @pl.when(h == nh - 1)
    def _fin():
        out_ref[0] = (out_ref[0] + b2_ref[0, 0][None, :]) * vals_ref[0, 0][:, None]


def _mlp(inp, w1a, b1, w2a, b2, vals):
    E, BK, D = inp.shape
    H = w1a.shape[2]
    O = w2a.shape[2]
    HB = 1024
    grid = (E, H // HB)
    w1a = w1a.astype(jnp.bfloat16)
    w2a = w2a.astype(jnp.bfloat16)
    return pl.pallas_call(
        _mlp_body,
        grid=grid,
        in_specs=[
            pl.BlockSpec((1, BK, D), lambda e, h: (e, 0, 0)),
            pl.BlockSpec((1, D, HB), lambda e, h: (e, 0, h)),
            pl.BlockSpec((1, 1, HB), lambda e, h: (e, 0, h)),
            pl.BlockSpec((1, HB, O), lambda e, h: (e, h, 0)),
            pl.BlockSpec((1, 1, O), lambda e, h: (e, 0, 0)),
            pl.BlockSpec((1, 1, BK), lambda e, h: (e, 0, 0)),
        ],
        out_specs=pl.BlockSpec((1, BK, O), lambda e, h: (e, 0, 0)),
        out_shape=jax.ShapeDtypeStruct((E, BK, O), jnp.float32),
        compiler_params=pltpu.CompilerParams(
            dimension_semantics=("parallel", "arbitrary"),
        ),
    )(inp, w1a, b1, w2a, b2, vals)


def kernel(x, gate_w, gate_b, weight1, weight2):
    B, S, D = x.shape
    E = weight1.shape[0]
    k = _K

    lg, m_s, iz_s = _gate(x, gate_w, gate_b)
    lg_r = lg.reshape(E * B, S)
    t_s, bud_s = _threshold(lg_r)
    sel_s, sel_p, inp = _compact(lg_r, t_s, bud_s,
                                 m_s.reshape(E * B, 16),
                                 iz_s.reshape(E * B, 16),
                                 x.reshape(B * S, D))
    inp = inp.reshape(E, B * k, D)
    valsE = sel_p.reshape(E, 1, B * k)

    w1a = weight1[:, :D, :]
    b1 = weight1[:, D:, :]
    w2a = weight2[:, :-1, :]
    b2 = weight2[:, -1:, :]

    out = _mlp(inp, w1a, b1, w2a, b2, valsE)

    O = out.shape[-1]
    return _scatter(out.reshape(E * B, k, O), sel_s, B, S)


# speedup vs baseline: 1.9148x; 1.0492x over previous
import functools

import jax
import jax.numpy as jnp
from jax import lax
from jax.experimental import pallas as pl
from jax.experimental.pallas import tpu as pltpu
from jax.experimental.pallas import tpu_sc as plsc

_K = 256


def _gate_body(x_ref, gw_ref, gb_ref, lg_ref, m_ref, iz_ref, m_scr, z_scr):
    s = pl.program_id(1)
    ns = pl.num_programs(1)
    xb = x_ref[0]
    gw = gw_ref[...]
    lg = lax.dot_general(gw, xb, (((1,), (1,)), ((), ())),
                         preferred_element_type=jnp.float32)
    lg = lg + gb_ref[:, :1]
    lg_ref[:, 0, 0, :] = lg
    bm = jnp.max(lg, axis=1, keepdims=True)
    bz = jnp.sum(jnp.exp(lg - bm), axis=1, keepdims=True)

    @pl.when(s == 0)
    def _init():
        m_scr[...] = jnp.broadcast_to(bm, m_scr.shape)
        z_scr[...] = jnp.broadcast_to(bz, z_scr.shape)

    @pl.when(s != 0)
    def _acc():
        m_old = m_scr[:, :1]
        z_old = z_scr[:, :1]
        m_new = jnp.maximum(m_old, bm)
        z_new = z_old * jnp.exp(m_old - m_new) + bz * jnp.exp(bm - m_new)
        m_scr[...] = jnp.broadcast_to(m_new, m_scr.shape)
        z_scr[...] = jnp.broadcast_to(z_new, z_scr.shape)

    @pl.when(s == ns - 1)
    def _fin():
        m_ref[:, 0, 0, :] = m_scr[...]
        iz_ref[:, 0, 0, :] = 1.0 / z_scr[...]


def _gate(x, gate_w, gate_b):
    B, S, D = x.shape
    E = gate_w.shape[0]
    SB = 1024
    grid = (B, S // SB)
    gb = jnp.broadcast_to(gate_b[:, None], (E, 16))
    return pl.pallas_call(
        _gate_body,
        grid=grid,
        in_specs=[
            pl.BlockSpec((1, SB, D), lambda b, s: (b, s, 0)),
            pl.BlockSpec((E, D), lambda b, s: (0, 0)),
            pl.BlockSpec((E, 16), lambda b, s: (0, 0)),
        ],
        out_specs=[
            pl.BlockSpec((E, 1, 1, SB), lambda b, s: (0, b, 0, s)),
            pl.BlockSpec((E, 1, 1, 16), lambda b, s: (0, b, 0, 0)),
            pl.BlockSpec((E, 1, 1, 16), lambda b, s: (0, b, 0, 0)),
        ],
        out_shape=[
            jax.ShapeDtypeStruct((E, B, 1, S), jnp.float32),
            jax.ShapeDtypeStruct((E, B, 1, 16), jnp.float32),
            jax.ShapeDtypeStruct((E, B, 1, 16), jnp.float32),
        ],
        scratch_shapes=[
            pltpu.VMEM((E, 16), jnp.float32),
            pltpu.VMEM((E, 16), jnp.float32),
        ],
        compiler_params=pltpu.CompilerParams(
            dimension_semantics=("arbitrary", "arbitrary"),
        ),
    )(x, gate_w, gb)


def _thresh_body(lg_ref, t_ref, bud_ref):
    int_min = lax.shift_left(jnp.int32(1), 31)
    bits = lax.bitcast_convert_type(lg_ref[...], jnp.int32)
    skey = bits ^ ((bits >> 31) & jnp.int32(0x7FFFFFFF))

    def step(i, u):
        bit = lax.shift_left(jnp.int32(1), 31 - i)
        ut = u | bit
        ts = ut ^ int_min
        cnt = jnp.sum((skey >= ts).astype(jnp.int32), axis=1, keepdims=True)
        return jnp.where(cnt >= _K, ut, u)

    u0 = jnp.zeros((lg_ref.shape[0], 1), jnp.int32)
    u = lax.fori_loop(0, 32, step, u0)
    ts = u ^ int_min
    g = jnp.sum((skey > ts).astype(jnp.int32), axis=1, keepdims=True)
    bits_t = jnp.where(u < 0, u ^ int_min, ~u)
    t_f = lax.bitcast_convert_type(bits_t, jnp.float32)
    t_ref[...] = jnp.broadcast_to(t_f, t_ref.shape)
    bud_ref[...] = jnp.broadcast_to(_K - g, bud_ref.shape)


def _threshold(lg_r):
    R, S = lg_r.shape
    RB = 8
    grid = (R // RB,)
    return pl.pallas_call(
        _thresh_body,
        grid=grid,
        in_specs=[pl.BlockSpec((RB, S), lambda i: (i, 0))],
        out_specs=[
            pl.BlockSpec((RB, 16), lambda i: (i, 0)),
            pl.BlockSpec((RB, 16), lambda i: (i, 0)),
        ],
        out_shape=[
            jax.ShapeDtypeStruct((R, 16), jnp.float32),
            jax.ShapeDtypeStruct((R, 16), jnp.int32),
        ],
        compiler_params=pltpu.CompilerParams(
            dimension_semantics=("arbitrary",),
        ),
    )(lg_r)


_NW = 32
_RPW = 256 // _NW


def _compact_body(lg, ts, bs, ms, zs, x2, sel_s, sel_p, inp,
                  row_v, t_v, b_v, m_v, z_v, os_v, ol_v, tie_v,
                  gi_v, rows_v, sem):
    cid = lax.axis_index("c")
    sid = lax.axis_index("s")
    wid = sid * 2 + cid

    for rr in range(_RPW):
        r = wid * _RPW + rr
        pltpu.sync_copy(lg.at[r], row_v)
        pltpu.sync_copy(ts.at[r], t_v)
        pltpu.sync_copy(bs.at[r], b_v)
        pltpu.sync_copy(ms.at[r], m_v)
        pltpu.sync_copy(zs.at[r], z_v)
        vt = t_v[...]
        vb = b_v[...]

        lanes = lax.iota(jnp.int32, 16)
        one = jnp.ones((16,), jnp.int32)
        zero = jnp.zeros((16,), jnp.int32)

        def step(c, carry):
            off_s, off_t, ii = carry
            v = row_v[pl.ds(c * 16, 16)]
            gt = v > vt
            eq = v == vt
            cs_g = plsc.cumsum(jnp.where(gt, one, zero))
            cs_e = plsc.cumsum(jnp.where(eq, one, zero))
            pos_g = off_s + cs_g - one
            pos_e = off_t + cs_e - one
            plsc.store_scatter(os_v, [pos_g], ii, mask=gt)
            plsc.store_scatter(ol_v, [pos_g], v, mask=gt)
            plsc.store_scatter(tie_v, [pos_e], ii, mask=eq)
            n_g = plsc.all_reduce_population_count(gt)
            n_e = plsc.all_reduce_population_count(eq)
            return off_s + n_g, off_t + n_e, ii + 16 * one

        off_s, _, _ = lax.fori_loop(
            0, 512, step, (zero, zero, lanes))

        nti = (jnp.max(vb) + 15) // 16

        def tstep(t, off):
            tv = tie_v[pl.ds(t * 16, 16)]
            tbase = jnp.full((16,), t * 16, jnp.int32)
            mk = (lanes + tbase) < vb
            cs = plsc.cumsum(jnp.where(mk, one, zero))
            pos = off + cs - one
            plsc.store_scatter(os_v, [pos], tv, mask=mk)
            plsc.store_scatter(ol_v, [pos], vt, mask=mk)
            return off + plsc.all_reduce_population_count(mk)

        lax.fori_loop(0, nti, tstep, off_s)

        vm = m_v[...]
        vz = z_v[...]
        voff = jnp.full((16,), (r % 4) * 8192, jnp.int32)
        for j in range(_K // 16):
            lv = ol_v[pl.ds(j * 16, 16)]
            ol_v[pl.ds(j * 16, 16)] = jnp.exp(lv - vm) * vz
            gi_v[pl.ds(j * 16, 16)] = os_v[pl.ds(j * 16, 16)] + voff

        pltpu.async_copy(x2.at[gi_v], rows_v, sem).wait()
        pltpu.sync_copy(rows_v, inp.at[r])
        pltpu.sync_copy(os_v.at[pl.ds(0, _K)], sel_s.at[r])
        pltpu.sync_copy(ol_v.at[pl.ds(0, _K)], sel_p.at[r])


def _compact(lg_r, t_splat, bud_splat, m_splat, iz_splat, x2):
    R, S = lg_r.shape
    D = x2.shape[1]
    mesh = plsc.VectorSubcoreMesh(core_axis_name="c", subcore_axis_name="s",
                                  num_cores=2, num_subcores=16)
    fn = pl.kernel(
        _compact_body,
        out_type=[
            jax.ShapeDtypeStruct((R, _K), jnp.int32),
            jax.ShapeDtypeStruct((R, _K), jnp.float32),
            jax.ShapeDtypeStruct((R, _K, D), jnp.float32),
        ],
        mesh=mesh,
        scratch_types=[
            pltpu.VMEM((S,), jnp.float32),
            pltpu.VMEM((16,), jnp.float32),
            pltpu.VMEM((16,), jnp.int32),
            pltpu.VMEM((16,), jnp.float32),
            pltpu.VMEM((16,), jnp.float32),
            pltpu.VMEM((_K + 16,), jnp.int32),
            pltpu.VMEM((_K + 16,), jnp.float32),
            pltpu.VMEM((S + 16,), jnp.int32),
            pltpu.VMEM((_K,), jnp.int32),
            pltpu.VMEM((_K, D), jnp.float32),
            pltpu.SemaphoreType.DMA,
        ],
        compiler_params=pltpu.CompilerParams(needs_layout_passes=False),
    )
    return fn(lg_r, t_splat, bud_splat, m_splat, iz_splat, x2)


def _scatter_body(rows, sel, zeros, out, ch_v, si_v, sh_v):
    cid = lax.axis_index("c")
    sid = lax.axis_index("s")

    for bb in range(2):
        batch = cid * 2 + bb
        pltpu.sync_copy(zeros.at[pl.ds(sid * 512, 512)],
                        sh_v.at[pl.ds(sid * 512, 512)])
        plsc.subcore_barrier()
        for i in range(4):
            e = sid + 16 * i
            r = e * 4 + batch
            pltpu.sync_copy(rows.at[r], ch_v)
            pltpu.sync_copy(sel.at[r], si_v)
            pltpu.sync_copy(ch_v, sh_v.at[si_v], add=True)
        plsc.subcore_barrier()
        pltpu.sync_copy(sh_v.at[pl.ds(sid * 512, 512)],
                        out.at[batch, pl.ds(sid * 512, 512)])
        plsc.subcore_barrier()


def _scatter(rows, sel, B, S):
    R, K, O = rows.shape
    mesh = plsc.VectorSubcoreMesh(core_axis_name="c", subcore_axis_name="s",
                                  num_cores=2, num_subcores=16)
    fn = pl.kernel(
        _scatter_body,
        out_type=jax.ShapeDtypeStruct((B, S, O), jnp.float32),
        mesh=mesh,
        scratch_types=[
            pltpu.VMEM((K, O), jnp.float32),
            pltpu.VMEM((K,), jnp.int32),
            pltpu.VMEM_SHARED((S, O), jnp.float32),
        ],
        compiler_params=pltpu.CompilerParams(needs_layout_passes=False),
    )
    return fn(rows, sel, jnp.zeros((S, O), jnp.float32))


def _mlp_body(inp_ref, w1_ref, b1_ref, w2_ref, b2_ref, vals_ref, out_ref):
    h = pl.program_id(1)
    nh = pl.num_programs(1)
    a = jnp.dot(inp_ref[0].astype(jnp.bfloat16), w1_ref[0],
                preferred_element_type=jnp.float32)
    a = a + b1_ref[0, 0][None, :]
    g = 0.5 * a * (1.0 + jax.lax.erf(a * 0.7071067811865476))
    part = jnp.dot(g.astype(jnp.bfloat16), w2_ref[0],
                   preferred_element_type=jnp.float32)

    @pl.when(h == 0)
    def _init():
        out_ref[0] = part

    @pl.when(h != 0)
    def _acc():
        out_ref[0] += part

    @pl.when(h == nh - 1)
    def _fin():
        out_ref[0] = (out_ref[0] + b2_ref[0, 0][None, :]) * vals_ref[0, 0][:, None]


def _mlp(inp, w1a, b1, w2a, b2, vals):
    E, BK, D = inp.shape
    H = w1a.shape[2]
    O = w2a.shape[2]
    HB = 2048
    grid = (E, H // HB)
    w1a = w1a.astype(jnp.bfloat16)
    w2a = w2a.astype(jnp.bfloat16)
    return pl.pallas_call(
        _mlp_body,
        grid=grid,
        in_specs=[
            pl.BlockSpec((1, BK, D), lambda e, h: (e, 0, 0)),
            pl.BlockSpec((1, D, HB), lambda e, h: (e, 0, h)),
            pl.BlockSpec((1, 1, HB), lambda e, h: (e, 0, h)),
            pl.BlockSpec((1, HB, O), lambda e, h: (e, h, 0)),
            pl.BlockSpec((1, 1, O), lambda e, h: (e, 0, 0)),
            pl.BlockSpec((1, 1, BK), lambda e, h: (e, 0, 0)),
        ],
        out_specs=pl.BlockSpec((1, BK, O), lambda e, h: (e, 0, 0)),
        out_shape=jax.ShapeDtypeStruct((E, BK, O), jnp.float32),
        compiler_params=pltpu.CompilerParams(
            dimension_semantics=("parallel", "arbitrary"),
        ),
    )(inp, w1a, b1, w2a, b2, vals)


def kernel(x, gate_w, gate_b, weight1, weight2):
    B, S, D = x.shape
    E = weight1.shape[0]
    k = _K

    lg, m_s, iz_s = _gate(x, gate_w, gate_b)
    lg_r = lg.reshape(E * B, S)
    t_s, bud_s = _threshold(lg_r)
    sel_s, sel_p, inp = _compact(lg_r, t_s, bud_s,
                                 m_s.reshape(E * B, 16),
                                 iz_s.reshape(E * B, 16),
                                 x.reshape(B * S, D))
    inp = inp.reshape(E, B * k, D)
    valsE = sel_p.reshape(E, 1, B * k)

    w1a = weight1[:, :D, :]
    b1 = weight1[:, D:, :]
    w2a = weight2[:, :-1, :]
    b2 = weight2[:, -1:, :]

    out = _mlp(inp, w1a, b1, w2a, b2, valsE)

    O = out.shape[-1]
    return _scatter(out.reshape(E * B, k, O), sel_s, B, S)


# speedup vs baseline: 2.0216x; 1.0558x over previous
import functools

import jax
import jax.numpy as jnp
from jax import lax
from jax.experimental import pallas as pl
from jax.experimental.pallas import tpu as pltpu
from jax.experimental.pallas import tpu_sc as plsc

_K = 256


def _gate_body(x_ref, gw_ref, gb_ref, lg_ref, m_ref, iz_ref, m_scr, z_scr):
    s = pl.program_id(1)
    ns = pl.num_programs(1)
    xb = x_ref[0]
    gw = gw_ref[...]
    lg = lax.dot_general(gw, xb, (((1,), (1,)), ((), ())),
                         preferred_element_type=jnp.float32)
    lg = lg + gb_ref[:, :1]
    lg_ref[:, 0, 0, :] = lg
    bm = jnp.max(lg, axis=1, keepdims=True)
    bz = jnp.sum(jnp.exp(lg - bm), axis=1, keepdims=True)

    @pl.when(s == 0)
    def _init():
        m_scr[...] = jnp.broadcast_to(bm, m_scr.shape)
        z_scr[...] = jnp.broadcast_to(bz, z_scr.shape)

    @pl.when(s != 0)
    def _acc():
        m_old = m_scr[:, :1]
        z_old = z_scr[:, :1]
        m_new = jnp.maximum(m_old, bm)
        z_new = z_old * jnp.exp(m_old - m_new) + bz * jnp.exp(bm - m_new)
        m_scr[...] = jnp.broadcast_to(m_new, m_scr.shape)
        z_scr[...] = jnp.broadcast_to(z_new, z_scr.shape)

    @pl.when(s == ns - 1)
    def _fin():
        m_ref[:, 0, 0, :] = m_scr[...]
        iz_ref[:, 0, 0, :] = 1.0 / z_scr[...]


def _gate(x, gate_w, gate_b):
    B, S, D = x.shape
    E = gate_w.shape[0]
    SB = 1024
    grid = (B, S // SB)
    gb = jnp.broadcast_to(gate_b[:, None], (E, 16))
    return pl.pallas_call(
        _gate_body,
        grid=grid,
        in_specs=[
            pl.BlockSpec((1, SB, D), lambda b, s: (b, s, 0)),
            pl.BlockSpec((E, D), lambda b, s: (0, 0)),
            pl.BlockSpec((E, 16), lambda b, s: (0, 0)),
        ],
        out_specs=[
            pl.BlockSpec((E, 1, 1, SB), lambda b, s: (0, b, 0, s)),
            pl.BlockSpec((E, 1, 1, 16), lambda b, s: (0, b, 0, 0)),
            pl.BlockSpec((E, 1, 1, 16), lambda b, s: (0, b, 0, 0)),
        ],
        out_shape=[
            jax.ShapeDtypeStruct((E, B, 1, S), jnp.float32),
            jax.ShapeDtypeStruct((E, B, 1, 16), jnp.float32),
            jax.ShapeDtypeStruct((E, B, 1, 16), jnp.float32),
        ],
        scratch_shapes=[
            pltpu.VMEM((E, 16), jnp.float32),
            pltpu.VMEM((E, 16), jnp.float32),
        ],
        compiler_params=pltpu.CompilerParams(
            dimension_semantics=("arbitrary", "arbitrary"),
        ),
    )(x, gate_w, gb)


def _thresh_body(lg_ref, t_ref, bud_ref):
    int_min = lax.shift_left(jnp.int32(1), 31)
    bits = lax.bitcast_convert_type(lg_ref[...], jnp.int32)
    skey = bits ^ ((bits >> 31) & jnp.int32(0x7FFFFFFF))

    def step(i, u):
        bit = lax.shift_left(jnp.int32(1), 31 - i)
        ut = u | bit
        ts = ut ^ int_min
        cnt = jnp.sum((skey >= ts).astype(jnp.int32), axis=1, keepdims=True)
        return jnp.where(cnt >= _K, ut, u)

    u0 = jnp.zeros((lg_ref.shape[0], 1), jnp.int32)
    u = lax.fori_loop(0, 32, step, u0)
    ts = u ^ int_min
    g = jnp.sum((skey > ts).astype(jnp.int32), axis=1, keepdims=True)
    bits_t = jnp.where(u < 0, u ^ int_min, ~u)
    t_f = lax.bitcast_convert_type(bits_t, jnp.float32)
    t_ref[...] = jnp.broadcast_to(t_f, t_ref.shape)
    bud_ref[...] = jnp.broadcast_to(_K - g, bud_ref.shape)


def _threshold(lg_r):
    R, S = lg_r.shape
    RB = 8
    grid = (R // RB,)
    return pl.pallas_call(
        _thresh_body,
        grid=grid,
        in_specs=[pl.BlockSpec((RB, S), lambda i: (i, 0))],
        out_specs=[
            pl.BlockSpec((RB, 16), lambda i: (i, 0)),
            pl.BlockSpec((RB, 16), lambda i: (i, 0)),
        ],
        out_shape=[
            jax.ShapeDtypeStruct((R, 16), jnp.float32),
            jax.ShapeDtypeStruct((R, 16), jnp.int32),
        ],
        compiler_params=pltpu.CompilerParams(
            dimension_semantics=("arbitrary",),
        ),
    )(lg_r)


_NW = 32
_RPW = 256 // _NW


def _compact_body(lg, ts, bs, ms, zs, x2, sel_s, sel_p, inp,
                  row_v, t_v, b_v, m_v, z_v, os_v, ol_v, tie_v,
                  gi_v, rows_v, sem):
    cid = lax.axis_index("c")
    sid = lax.axis_index("s")
    wid = sid * 2 + cid

    for rr in range(_RPW):
        r = wid * _RPW + rr
        pltpu.sync_copy(lg.at[r], row_v)
        pltpu.sync_copy(ts.at[r], t_v)
        pltpu.sync_copy(bs.at[r], b_v)
        pltpu.sync_copy(ms.at[r], m_v)
        pltpu.sync_copy(zs.at[r], z_v)
        vt = t_v[...]
        vb = b_v[...]

        lanes = lax.iota(jnp.int32, 16)
        one = jnp.ones((16,), jnp.int32)
        zero = jnp.zeros((16,), jnp.int32)

        def step(c, carry):
            off_s, off_t, ii = carry
            v = row_v[pl.ds(c * 16, 16)]
            gt = v > vt
            eq = v == vt
            cs_g = plsc.cumsum(jnp.where(gt, one, zero))
            cs_e = plsc.cumsum(jnp.where(eq, one, zero))
            pos_g = off_s + cs_g - one
            pos_e = off_t + cs_e - one
            plsc.store_scatter(os_v, [pos_g], ii, mask=gt)
            plsc.store_scatter(ol_v, [pos_g], v, mask=gt)
            plsc.store_scatter(tie_v, [pos_e], ii, mask=eq)
            n_g = plsc.all_reduce_population_count(gt)
            n_e = plsc.all_reduce_population_count(eq)
            return off_s + n_g, off_t + n_e, ii + 16 * one

        off_s, _, _ = lax.fori_loop(
            0, 512, step, (zero, zero, lanes))

        nti = (jnp.max(vb) + 15) // 16

        def tstep(t, off):
            tv = tie_v[pl.ds(t * 16, 16)]
            tbase = jnp.full((16,), t * 16, jnp.int32)
            mk = (lanes + tbase) < vb
            cs = plsc.cumsum(jnp.where(mk, one, zero))
            pos = off + cs - one
            plsc.store_scatter(os_v, [pos], tv, mask=mk)
            plsc.store_scatter(ol_v, [pos], vt, mask=mk)
            return off + plsc.all_reduce_population_count(mk)

        lax.fori_loop(0, nti, tstep, off_s)

        vm = m_v[...]
        vz = z_v[...]
        voff = jnp.full((16,), (r % 4) * 8192, jnp.int32)
        for j in range(_K // 16):
            lv = ol_v[pl.ds(j * 16, 16)]
            ol_v[pl.ds(j * 16, 16)] = jnp.exp(lv - vm) * vz
            gi_v[pl.ds(j * 16, 16)] = os_v[pl.ds(j * 16, 16)] + voff

        pltpu.async_copy(x2.at[gi_v], rows_v, sem).wait()
        pltpu.sync_copy(rows_v, inp.at[r])
        pltpu.sync_copy(os_v.at[pl.ds(0, _K)], sel_s.at[r])
        pltpu.sync_copy(ol_v.at[pl.ds(0, _K)], sel_p.at[r])


def _compact(lg_r, t_splat, bud_splat, m_splat, iz_splat, x2):
    R, S = lg_r.shape
    D = x2.shape[1]
    mesh = plsc.VectorSubcoreMesh(core_axis_name="c", subcore_axis_name="s",
                                  num_cores=2, num_subcores=16)
    fn = pl.kernel(
        _compact_body,
        out_type=[
            jax.ShapeDtypeStruct((R, _K), jnp.int32),
            jax.ShapeDtypeStruct((R, _K), jnp.float32),
            jax.ShapeDtypeStruct((R, _K, D), jnp.float32),
        ],
        mesh=mesh,
        scratch_types=[
            pltpu.VMEM((S,), jnp.float32),
            pltpu.VMEM((16,), jnp.float32),
            pltpu.VMEM((16,), jnp.int32),
            pltpu.VMEM((16,), jnp.float32),
            pltpu.VMEM((16,), jnp.float32),
            pltpu.VMEM((_K + 16,), jnp.int32),
            pltpu.VMEM((_K + 16,), jnp.float32),
            pltpu.VMEM((S + 16,), jnp.int32),
            pltpu.VMEM((_K,), jnp.int32),
            pltpu.VMEM((_K, D), jnp.float32),
            pltpu.SemaphoreType.DMA,
        ],
        compiler_params=pltpu.CompilerParams(needs_layout_passes=False),
    )
    return fn(lg_r, t_splat, bud_splat, m_splat, iz_splat, x2)


def _scatter_body(rows, sel, zeros, out, ch_v, si_v, sh_v):
    cid = lax.axis_index("c")
    sid = lax.axis_index("s")

    for bb in range(2):
        batch = cid * 2 + bb
        pltpu.sync_copy(zeros.at[pl.ds(sid * 512, 512)],
                        sh_v.at[pl.ds(sid * 512, 512)])
        plsc.subcore_barrier()
        for i in range(4):
            e = sid + 16 * i
            r = e * 4 + batch
            pltpu.sync_copy(rows.at[r], ch_v)
            pltpu.sync_copy(sel.at[r], si_v)
            pltpu.sync_copy(ch_v, sh_v.at[si_v], add=True)
        plsc.subcore_barrier()
        pltpu.sync_copy(sh_v.at[pl.ds(sid * 512, 512)],
                        out.at[batch, pl.ds(sid * 512, 512)])
        plsc.subcore_barrier()


def _scatter(rows, sel, B, S):
    R, K, O = rows.shape
    mesh = plsc.VectorSubcoreMesh(core_axis_name="c", subcore_axis_name="s",
                                  num_cores=2, num_subcores=16)
    fn = pl.kernel(
        _scatter_body,
        out_type=jax.ShapeDtypeStruct((B, S, O), jnp.float32),
        mesh=mesh,
        scratch_types=[
            pltpu.VMEM((K, O), jnp.float32),
            pltpu.VMEM((K,), jnp.int32),
            pltpu.VMEM_SHARED((S, O), jnp.float32),
        ],
        compiler_params=pltpu.CompilerParams(needs_layout_passes=False),
    )
    return fn(rows, sel, jnp.zeros((S, O), jnp.float32))


def _mlp_body(inp_ref, w1_ref, w2_ref, vals_ref, out_ref):
    a = jnp.dot(inp_ref[0].astype(jnp.bfloat16),
                w1_ref[0, :-1, :].astype(jnp.bfloat16),
                preferred_element_type=jnp.float32)
    a = a + w1_ref[0, -1:, :]
    g = 0.5 * a * (1.0 + jax.lax.erf(a * 0.7071067811865476))
    part = jnp.dot(g.astype(jnp.bfloat16),
                   w2_ref[0, :-1, :].astype(jnp.bfloat16),
                   preferred_element_type=jnp.float32)
    out_ref[0] = (part + w2_ref[0, -1:, :]) * vals_ref[0, 0][:, None]


def _mlp(inp, w1, w2, vals):
    E, BK, D = inp.shape
    H = w1.shape[2]
    O = w2.shape[2]
    grid = (E,)
    return pl.pallas_call(
        _mlp_body,
        grid=grid,
        in_specs=[
            pl.BlockSpec((1, BK, D), lambda e: (e, 0, 0)),
            pl.BlockSpec((1, D + 1, H), lambda e: (e, 0, 0)),
            pl.BlockSpec((1, H + 1, O), lambda e: (e, 0, 0)),
            pl.BlockSpec((1, 1, BK), lambda e: (e, 0, 0)),
        ],
        out_specs=pl.BlockSpec((1, BK, O), lambda e: (e, 0, 0)),
        out_shape=jax.ShapeDtypeStruct((E, BK, O), jnp.float32),
        compiler_params=pltpu.CompilerParams(
            dimension_semantics=("parallel",),
        ),
    )(inp, w1, w2, vals)


def kernel(x, gate_w, gate_b, weight1, weight2):
    B, S, D = x.shape
    E = weight1.shape[0]
    k = _K

    lg, m_s, iz_s = _gate(x, gate_w, gate_b)
    lg_r = lg.reshape(E * B, S)
    t_s, bud_s = _threshold(lg_r)
    sel_s, sel_p, inp = _compact(lg_r, t_s, bud_s,
                                 m_s.reshape(E * B, 16),
                                 iz_s.reshape(E * B, 16),
                                 x.reshape(B * S, D))
    inp = inp.reshape(E, B * k, D)
    valsE = sel_p.reshape(E, 1, B * k)

    out = _mlp(inp, weight1, weight2, valsE)

    O = out.shape[-1]
    return _scatter(out.reshape(E * B, k, O), sel_s, B, S)


# speedup vs baseline: 2.0798x; 1.0288x over previous
import functools

import jax
import jax.numpy as jnp
from jax import lax
from jax.experimental import pallas as pl
from jax.experimental.pallas import tpu as pltpu
from jax.experimental.pallas import tpu_sc as plsc

_K = 256


def _gate_body(x_ref, gw_ref, gb_ref, lg_ref, m_ref, iz_ref, m_scr, z_scr):
    s = pl.program_id(1)
    ns = pl.num_programs(1)
    xb = x_ref[0]
    gw = gw_ref[...]
    lg = lax.dot_general(gw, xb, (((1,), (1,)), ((), ())),
                         preferred_element_type=jnp.float32)
    lg = lg + gb_ref[:, :1]
    lg_ref[:, 0, 0, :] = lg
    bm = jnp.max(lg, axis=1, keepdims=True)
    bz = jnp.sum(jnp.exp(lg - bm), axis=1, keepdims=True)

    @pl.when(s == 0)
    def _init():
        m_scr[...] = jnp.broadcast_to(bm, m_scr.shape)
        z_scr[...] = jnp.broadcast_to(bz, z_scr.shape)

    @pl.when(s != 0)
    def _acc():
        m_old = m_scr[:, :1]
        z_old = z_scr[:, :1]
        m_new = jnp.maximum(m_old, bm)
        z_new = z_old * jnp.exp(m_old - m_new) + bz * jnp.exp(bm - m_new)
        m_scr[...] = jnp.broadcast_to(m_new, m_scr.shape)
        z_scr[...] = jnp.broadcast_to(z_new, z_scr.shape)

    @pl.when(s == ns - 1)
    def _fin():
        m_ref[:, 0, 0, :] = m_scr[...]
        iz_ref[:, 0, 0, :] = 1.0 / z_scr[...]


def _gate(x, gate_w, gate_b):
    B, S, D = x.shape
    E = gate_w.shape[0]
    SB = 1024
    grid = (B, S // SB)
    gb = jnp.broadcast_to(gate_b[:, None], (E, 16))
    return pl.pallas_call(
        _gate_body,
        grid=grid,
        in_specs=[
            pl.BlockSpec((1, SB, D), lambda b, s: (b, s, 0)),
            pl.BlockSpec((E, D), lambda b, s: (0, 0)),
            pl.BlockSpec((E, 16), lambda b, s: (0, 0)),
        ],
        out_specs=[
            pl.BlockSpec((E, 1, 1, SB), lambda b, s: (0, b, 0, s)),
            pl.BlockSpec((E, 1, 1, 16), lambda b, s: (0, b, 0, 0)),
            pl.BlockSpec((E, 1, 1, 16), lambda b, s: (0, b, 0, 0)),
        ],
        out_shape=[
            jax.ShapeDtypeStruct((E, B, 1, S), jnp.float32),
            jax.ShapeDtypeStruct((E, B, 1, 16), jnp.float32),
            jax.ShapeDtypeStruct((E, B, 1, 16), jnp.float32),
        ],
        scratch_shapes=[
            pltpu.VMEM((E, 16), jnp.float32),
            pltpu.VMEM((E, 16), jnp.float32),
        ],
        compiler_params=pltpu.CompilerParams(
            dimension_semantics=("arbitrary", "arbitrary"),
        ),
    )(x, gate_w, gb)


def _thresh_body(lg_ref, t_ref, bud_ref):
    int_min = lax.shift_left(jnp.int32(1), 31)
    bits = lax.bitcast_convert_type(lg_ref[...], jnp.int32)
    skey = bits ^ ((bits >> 31) & jnp.int32(0x7FFFFFFF))

    def step(i, u):
        bit = lax.shift_left(jnp.int32(1), 31 - i)
        ut = u | bit
        ts = ut ^ int_min
        cnt = jnp.sum((skey >= ts).astype(jnp.int32), axis=1, keepdims=True)
        return jnp.where(cnt >= _K, ut, u)

    u0 = jnp.zeros((lg_ref.shape[0], 1), jnp.int32)
    u = lax.fori_loop(0, 32, step, u0)
    ts = u ^ int_min
    g = jnp.sum((skey > ts).astype(jnp.int32), axis=1, keepdims=True)
    bits_t = jnp.where(u < 0, u ^ int_min, ~u)
    t_f = lax.bitcast_convert_type(bits_t, jnp.float32)
    t_ref[...] = jnp.broadcast_to(t_f, t_ref.shape)
    bud_ref[...] = jnp.broadcast_to(_K - g, bud_ref.shape)


def _threshold(lg_r):
    R, S = lg_r.shape
    RB = 8
    grid = (R // RB,)
    return pl.pallas_call(
        _thresh_body,
        grid=grid,
        in_specs=[pl.BlockSpec((RB, S), lambda i: (i, 0))],
        out_specs=[
            pl.BlockSpec((RB, 16), lambda i: (i, 0)),
            pl.BlockSpec((RB, 16), lambda i: (i, 0)),
        ],
        out_shape=[
            jax.ShapeDtypeStruct((R, 16), jnp.float32),
            jax.ShapeDtypeStruct((R, 16), jnp.int32),
        ],
        compiler_params=pltpu.CompilerParams(
            dimension_semantics=("arbitrary",),
        ),
    )(lg_r)


_NW = 32
_RPW = 256 // _NW


def _compact_body(lg, ts, bs, ms, zs, x2, sel_s, sel_p, inp,
                  row_v, t_v, b_v, m_v, z_v, os_v, ol_v, os2_v, ol2_v,
                  gi_v, rows_v, sem):
    cid = lax.axis_index("c")
    sid = lax.axis_index("s")
    wid = sid * 2 + cid

    for rr in range(_RPW):
        r = wid * _RPW + rr
        pltpu.sync_copy(lg.at[r], row_v)
        pltpu.sync_copy(ts.at[r], t_v)
        pltpu.sync_copy(bs.at[r], b_v)
        pltpu.sync_copy(ms.at[r], m_v)
        pltpu.sync_copy(zs.at[r], z_v)
        vt = t_v[...]
        vb = b_v[...]

        lanes = lax.iota(jnp.int32, 16)
        one = jnp.ones((16,), jnp.int32)
        zero = jnp.zeros((16,), jnp.int32)

        def step(c, carry):
            off, ii = carry
            v0 = row_v[pl.ds(c * 32, 16)]
            v1 = row_v[pl.ds(c * 32 + 16, 16)]
            ge0 = v0 >= vt
            ge1 = v1 >= vt
            cs0 = plsc.cumsum(jnp.where(ge0, one, zero))
            cs1 = plsc.cumsum(jnp.where(ge1, one, zero))
            n0 = plsc.all_reduce_population_count(ge0)
            n1 = plsc.all_reduce_population_count(ge1)
            pos0 = off + cs0 - one
            plsc.store_scatter(os_v, [pos0], ii, mask=ge0)
            plsc.store_scatter(ol_v, [pos0], v0, mask=ge0)
            off1 = off + n0
            pos1 = off1 + cs1 - one
            ii1 = ii + 16 * one
            plsc.store_scatter(os_v, [pos1], ii1, mask=ge1)
            plsc.store_scatter(ol_v, [pos1], v1, mask=ge1)
            return off1 + n1, ii + 32 * one

        nsel, _ = lax.fori_loop(0, 256, step, (zero, lanes))

        ntr = (jnp.max(nsel) + 15) // 16

        def fstep(t, carry):
            off2, tseen = carry
            tbase = jnp.full((16,), t * 16, jnp.int32)
            valid = (lanes + tbase) < nsel
            iv = os_v[pl.ds(t * 16, 16)]
            lv = ol_v[pl.ds(t * 16, 16)]
            eq = (lv == vt) & valid
            gt = (lv > vt) & valid
            tr = plsc.cumsum(jnp.where(eq, one, zero)) + tseen
            keep = gt | (eq & (tr <= vb))
            cs = plsc.cumsum(jnp.where(keep, one, zero))
            pos = off2 + cs - one
            plsc.store_scatter(os2_v, [pos], iv, mask=keep)
            plsc.store_scatter(ol2_v, [pos], lv, mask=keep)
            return (off2 + plsc.all_reduce_population_count(keep),
                    tseen + plsc.all_reduce_population_count(eq))

        lax.fori_loop(0, ntr, fstep, (zero, zero))

        vm = m_v[...]
        vz = z_v[...]
        voff = jnp.full((16,), (r % 4) * 8192, jnp.int32)
        for j in range(_K // 16):
            lv = ol2_v[pl.ds(j * 16, 16)]
            ol2_v[pl.ds(j * 16, 16)] = jnp.exp(lv - vm) * vz
            gi_v[pl.ds(j * 16, 16)] = os2_v[pl.ds(j * 16, 16)] + voff

        pltpu.async_copy(x2.at[gi_v], rows_v, sem).wait()
        pltpu.sync_copy(rows_v, inp.at[r])
        pltpu.sync_copy(os2_v.at[pl.ds(0, _K)], sel_s.at[r])
        pltpu.sync_copy(ol2_v.at[pl.ds(0, _K)], sel_p.at[r])


def _compact(lg_r, t_splat, bud_splat, m_splat, iz_splat, x2):
    R, S = lg_r.shape
    D = x2.shape[1]
    mesh = plsc.VectorSubcoreMesh(core_axis_name="c", subcore_axis_name="s",
                                  num_cores=2, num_subcores=16)
    fn = pl.kernel(
        _compact_body,
        out_type=[
            jax.ShapeDtypeStruct((R, _K), jnp.int32),
            jax.ShapeDtypeStruct((R, _K), jnp.float32),
            jax.ShapeDtypeStruct((R, _K, D), jnp.float32),
        ],
        mesh=mesh,
        scratch_types=[
            pltpu.VMEM((S,), jnp.float32),
            pltpu.VMEM((16,), jnp.float32),
            pltpu.VMEM((16,), jnp.int32),
            pltpu.VMEM((16,), jnp.float32),
            pltpu.VMEM((16,), jnp.float32),
            pltpu.VMEM((S + 16,), jnp.int32),
            pltpu.VMEM((S + 16,), jnp.float32),
            pltpu.VMEM((_K + 16,), jnp.int32),
            pltpu.VMEM((_K + 16,), jnp.float32),
            pltpu.VMEM((_K,), jnp.int32),
            pltpu.VMEM((_K, D), jnp.float32),
            pltpu.SemaphoreType.DMA,
        ],
        compiler_params=pltpu.CompilerParams(needs_layout_passes=False),
    )
    return fn(lg_r, t_splat, bud_splat, m_splat, iz_splat, x2)


def _scatter_body(rows, sel, zeros, out, ch_v, si_v, sh_v):
    cid = lax.axis_index("c")
    sid = lax.axis_index("s")

    for bb in range(2):
        batch = cid * 2 + bb
        pltpu.sync_copy(zeros.at[pl.ds(sid * 512, 512)],
                        sh_v.at[pl.ds(sid * 512, 512)])
        plsc.subcore_barrier()
        for i in range(4):
            e = sid + 16 * i
            r = e * 4 + batch
            pltpu.sync_copy(rows.at[r], ch_v)
            pltpu.sync_copy(sel.at[r], si_v)
            pltpu.sync_copy(ch_v, sh_v.at[si_v], add=True)
        plsc.subcore_barrier()
        pltpu.sync_copy(sh_v.at[pl.ds(sid * 512, 512)],
                        out.at[batch, pl.ds(sid * 512, 512)])
        plsc.subcore_barrier()


def _scatter(rows, sel, B, S):
    R, K, O = rows.shape
    mesh = plsc.VectorSubcoreMesh(core_axis_name="c", subcore_axis_name="s",
                                  num_cores=2, num_subcores=16)
    fn = pl.kernel(
        _scatter_body,
        out_type=jax.ShapeDtypeStruct((B, S, O), jnp.float32),
        mesh=mesh,
        scratch_types=[
            pltpu.VMEM((K, O), jnp.float32),
            pltpu.VMEM((K,), jnp.int32),
            pltpu.VMEM_SHARED((S, O), jnp.float32),
        ],
        compiler_params=pltpu.CompilerParams(needs_layout_passes=False),
    )
    return fn(rows, sel, jnp.zeros((S, O), jnp.float32))


def _mlp_body(inp_ref, w1_ref, w2_ref, vals_ref, out_ref):
    a = jnp.dot(inp_ref[0].astype(jnp.bfloat16),
                w1_ref[0, :-1, :].astype(jnp.bfloat16),
                preferred_element_type=jnp.float32)
    a = a + w1_ref[0, -1:, :]
    g = 0.5 * a * (1.0 + jax.lax.erf(a * 0.7071067811865476))
    part = jnp.dot(g.astype(jnp.bfloat16),
                   w2_ref[0, :-1, :].astype(jnp.bfloat16),
                   preferred_element_type=jnp.float32)
    out_ref[0] = (part + w2_ref[0, -1:, :]) * vals_ref[0, 0][:, None]


def _mlp(inp, w1, w2, vals):
    E, BK, D = inp.shape
    H = w1.shape[2]
    O = w2.shape[2]
    grid = (E,)
    return pl.pallas_call(
        _mlp_body,
        grid=grid,
        in_specs=[
            pl.BlockSpec((1, BK, D), lambda e: (e, 0, 0)),
            pl.BlockSpec((1, D + 1, H), lambda e: (e, 0, 0)),
            pl.BlockSpec((1, H + 1, O), lambda e: (e, 0, 0)),
            pl.BlockSpec((1, 1, BK), lambda e: (e, 0, 0)),
        ],
        out_specs=pl.BlockSpec((1, BK, O), lambda e: (e, 0, 0)),
        out_shape=jax.ShapeDtypeStruct((E, BK, O), jnp.float32),
        compiler_params=pltpu.CompilerParams(
            dimension_semantics=("parallel",),
        ),
    )(inp, w1, w2, vals)


def kernel(x, gate_w, gate_b, weight1, weight2):
    B, S, D = x.shape
    E = weight1.shape[0]
    k = _K

    lg, m_s, iz_s = _gate(x, gate_w, gate_b)
    lg_r = lg.reshape(E * B, S)
    t_s, bud_s = _threshold(lg_r)
    sel_s, sel_p, inp = _compact(lg_r, t_s, bud_s,
                                 m_s.reshape(E * B, 16),
                                 iz_s.reshape(E * B, 16),
                                 x.reshape(B * S, D))
    inp = inp.reshape(E, B * k, D)
    valsE = sel_p.reshape(E, 1, B * k)

    out = _mlp(inp, weight1, weight2, valsE)

    O = out.shape[-1]
    return _scatter(out.reshape(E * B, k, O), sel_s, B, S)
